# Initial kernel scaffold; baseline (speedup 1.0000x reference)
import jax
import jax.numpy as jnp
from jax.experimental import pallas as pl


def kernel(x1, edge_index1, batch1, x2, edge_index2, batch2, W1, att_src1, att_dst1, b1, W2, att_src2, att_dst2, b2):
    raise NotImplementedError("write your pallas kernel here")


# speedup vs baseline: 11.6576x; 11.6576x over previous
import functools

import jax
import jax.numpy as jnp
from jax import lax
from jax.experimental import pallas as pl
from jax.experimental.pallas import tpu as pltpu
from jax.experimental.pallas import tpu_sc as plsc

_N = 10000
_D = 128
_H = 64
_HEADS = 4
_G = 16

_NP = 10240
_NB = 512
_NBN = _NP // _NB
_CH = 80
_EB = 128
_E = 320000
_ET = _E + _N
_EPB = 2816
_EPAD = _EPB * _EB
_NEG = -1e30

_NC = 2
_NS = 16


def _elu(v):
    return jnp.where(v > 0, v, jnp.exp(jnp.minimum(v, 0.0)) - 1.0)


def _tc1_body(x_ref, w_ref, ast_ref, adt_ref, htab_ref, as_ref, ad_ref, sm_ref):
    i = pl.program_id(0)
    h = jnp.dot(x_ref[...], w_ref[...], preferred_element_type=jnp.float32)
    asT = lax.dot_general(ast_ref[...], h, (((1,), (1,)), ((), ())),
                          preferred_element_type=jnp.float32)
    adT = lax.dot_general(adt_ref[...], h, (((1,), (1,)), ((), ())),
                          preferred_element_type=jnp.float32)
    gidx = i * _NB + lax.broadcasted_iota(jnp.int32, (8, _NB), 1)
    valid = gidx < _N
    asT = jnp.where(valid, asT, _NEG)
    adT = jnp.where(valid, adT, 0.0)
    as_ref[...] = asT
    ad_ref[...] = adT
    tail = (lax.broadcasted_iota(jnp.int32, (_NB, _CH - _H), 1) == 0)
    tail = tail.astype(jnp.float32)
    for hd in range(_HEADS):
        htab_ref[hd, :, 0:_H] = h[:, hd * _H:(hd + 1) * _H]
        htab_ref[hd, :, _H:_CH] = tail
    sa = jnp.max(asT, axis=1)
    sd = jnp.max(adT, axis=1)
    cur = jnp.concatenate([jnp.broadcast_to(sa[:, None], (8, 64)),
                           jnp.broadcast_to(sd[:, None], (8, 64))], axis=1)

    @pl.when(i == 0)
    def _():
        sm_ref[...] = cur

    @pl.when(i > 0)
    def _():
        sm_ref[...] = jnp.maximum(sm_ref[...], cur)


_tc1 = pl.pallas_call(
    _tc1_body,
    grid=(_NBN,),
    in_specs=[
        pl.BlockSpec((_NB, _D), lambda i: (i, 0)),
        pl.BlockSpec((_D, _HEADS * _H), lambda i: (0, 0)),
        pl.BlockSpec((8, _HEADS * _H), lambda i: (0, 0)),
        pl.BlockSpec((8, _HEADS * _H), lambda i: (0, 0)),
    ],
    out_specs=[
        pl.BlockSpec((_HEADS, _NB, _CH), lambda i: (0, i, 0)),
        pl.BlockSpec((8, _NB), lambda i: (0, i)),
        pl.BlockSpec((8, _NB), lambda i: (0, i)),
        pl.BlockSpec((8, 128), lambda i: (0, 0)),
    ],
    out_shape=[
        jax.ShapeDtypeStruct((_HEADS, _NP, _CH), jnp.float32),
        jax.ShapeDtypeStruct((8, _NP), jnp.float32),
        jax.ShapeDtypeStruct((8, _NP), jnp.float32),
        jax.ShapeDtypeStruct((8, 128), jnp.float32),
    ],
)


def _tc2_body(acca_ref, accb_ref, b1_ref, w2_ref, ast_ref, adt_ref,
              htab_ref, as_ref, ad_ref, sm_ref):
    i = pl.program_id(0)
    xs = []
    for hd in range(_HEADS):
        blk = acca_ref[hd] if hd < 2 else accb_ref[hd - 2]
        num = blk[:, 0:_H]
        den = blk[:, _H:_H + 1]
        v = num / (den + 1e-16) + b1_ref[0:1, hd * _H:(hd + 1) * _H]
        xs.append(_elu(v))
    x2 = jnp.concatenate(xs, axis=1)
    h2 = jnp.dot(x2, w2_ref[...], preferred_element_type=jnp.float32)
    asT = lax.dot_general(ast_ref[...], h2, (((1,), (1,)), ((), ())),
                          preferred_element_type=jnp.float32)
    adT = lax.dot_general(adt_ref[...], h2, (((1,), (1,)), ((), ())),
                          preferred_element_type=jnp.float32)
    gidx = i * _NB + lax.broadcasted_iota(jnp.int32, (8, _NB), 1)
    valid = gidx < _N
    asT = jnp.where(valid, asT, _NEG)
    adT = jnp.where(valid, adT, 0.0)
    as_ref[...] = asT
    ad_ref[...] = adT
    tail = (lax.broadcasted_iota(jnp.int32, (_NB, _CH - _H), 1) == 0)
    htab_ref[:, 0:_H] = h2
    htab_ref[:, _H:_CH] = tail.astype(jnp.float32)
    sa = jnp.max(asT, axis=1)
    sd = jnp.max(adT, axis=1)
    cur = jnp.concatenate([jnp.broadcast_to(sa[:, None], (8, 64)),
                           jnp.broadcast_to(sd[:, None], (8, 64))], axis=1)

    @pl.when(i == 0)
    def _():
        sm_ref[...] = cur

    @pl.when(i > 0)
    def _():
        sm_ref[...] = jnp.maximum(sm_ref[...], cur)


_tc2 = pl.pallas_call(
    _tc2_body,
    grid=(_NBN,),
    in_specs=[
        pl.BlockSpec((2, _NB, _CH), lambda i: (0, i, 0)),
        pl.BlockSpec((2, _NB, _CH), lambda i: (0, i, 0)),
        pl.BlockSpec((1, _HEADS * _H), lambda i: (0, 0)),
        pl.BlockSpec((_HEADS * _H, _H), lambda i: (0, 0)),
        pl.BlockSpec((8, _H), lambda i: (0, 0)),
        pl.BlockSpec((8, _H), lambda i: (0, 0)),
    ],
    out_specs=[
        pl.BlockSpec((_NB, _CH), lambda i: (i, 0)),
        pl.BlockSpec((8, _NB), lambda i: (0, i)),
        pl.BlockSpec((8, _NB), lambda i: (0, i)),
        pl.BlockSpec((8, 128), lambda i: (0, 0)),
    ],
    out_shape=[
        jax.ShapeDtypeStruct((_NP, _CH), jnp.float32),
        jax.ShapeDtypeStruct((8, _NP), jnp.float32),
        jax.ShapeDtypeStruct((8, _NP), jnp.float32),
        jax.ShapeDtypeStruct((8, 128), jnp.float32),
    ],
)


def _tc3_body(acc_ref, b2_ref, batch_ref, po_ref):
    i = pl.program_id(0)
    num = acc_ref[0, :, 0:_H] + acc_ref[1, :, 0:_H]
    den = acc_ref[0, :, _H:_H + 1] + acc_ref[1, :, _H:_H + 1]
    o = _elu(num / (den + 1e-16) + b2_ref[0:1, :])
    tail = (lax.broadcasted_iota(jnp.int32, (_NB, 64), 1) == 0)
    oa = jnp.concatenate([o, tail.astype(jnp.float32)], axis=1)
    bt = batch_ref[0, 0, :]
    P = (lax.broadcasted_iota(jnp.int32, (_G, _NB), 0)
         == bt[None, :]).astype(jnp.float32)
    contrib = jnp.dot(P, oa, preferred_element_type=jnp.float32)

    @pl.when(i == 0)
    def _():
        po_ref[...] = contrib

    @pl.when(i > 0)
    def _():
        po_ref[...] = po_ref[...] + contrib

    @pl.when(i == _NBN - 1)
    def _():
        s = po_ref[...]
        cnt = jnp.maximum(s[:, _H:_H + 1], 1.0)
        po_ref[...] = s / cnt


_tc3 = pl.pallas_call(
    _tc3_body,
    grid=(_NBN,),
    in_specs=[
        pl.BlockSpec((2, _NB, _CH), lambda i: (0, i, 0)),
        pl.BlockSpec((1, _H), lambda i: (0, 0)),
        pl.BlockSpec((1, 1, _NB), lambda i: (i, 0, 0)),
    ],
    out_specs=pl.BlockSpec((_G, 128), lambda i: (0, 0)),
    out_shape=jax.ShapeDtypeStruct((_G, 128), jnp.float32),
)


_SB = 8


def _make_sc_layer(head_offset, blocks_per_tile, edge_split):
    rows_per_tile = _NP // _NS
    nsb = blocks_per_tile // _SB
    mesh = plsc.VectorSubcoreMesh(core_axis_name="c", subcore_axis_name="s",
                                  num_cores=_NC, num_subcores=_NS)

    @functools.partial(
        pl.kernel,
        out_type=jax.ShapeDtypeStruct((2 * _NP, _CH), jnp.float32),
        mesh=mesh,
        compiler_params=pltpu.CompilerParams(needs_layout_passes=False,
                                             use_tc_tiling_on_sc=False),
        scratch_types=[
            pltpu.VMEM((_SB, _EB), jnp.int32),
            pltpu.VMEM((_SB, _EB), jnp.int32),
            pltpu.VMEM((_NP,), jnp.float32),
            pltpu.VMEM((_NP,), jnp.float32),
            pltpu.VMEM((1024,), jnp.float32),
            pltpu.VMEM((_EB, _CH), jnp.float32),
            pltpu.VMEM((_EB,), jnp.float32),
            pltpu.VMEM((1, _EB), jnp.int32),
            pltpu.VMEM((1, _EB), jnp.int32),
            pltpu.VMEM_SHARED((_NP, _CH), jnp.float32),
            pltpu.SemaphoreType.DMA,
        ],
    )
    def sck(src_hbm, dst_hbm, ast_hbm, adt_hbm, sm_hbm, htab_hbm, out_hbm,
            src_v, dst_v, as_v, ad_v, sm_v, rows_v, w_v, gi_v, si_v,
            acc_s, sem):
        c = lax.axis_index("c")
        s = lax.axis_index("s")
        hg = head_offset if edge_split else head_offset + c

        def _zb(b, carry):
            for cc in range(_CH // 16):
                rows_v[b, pl.ds(cc * 16, 16)] = jnp.zeros((16,), jnp.float32)
            return carry
        lax.fori_loop(0, _EB, _zb, 0)
        for z in range(rows_per_tile // _EB):
            pltpu.sync_copy(
                rows_v, acc_s.at[pl.ds(s * rows_per_tile + z * _EB, _EB)])

        pltpu.sync_copy(ast_hbm.at[hg], as_v)
        pltpu.sync_copy(adt_hbm.at[hg], ad_v)
        pltpu.sync_copy(sm_hbm, sm_v)
        plsc.subcore_barrier()

        lanes = lax.iota(jnp.int32, 16)
        sa = plsc.load_gather(sm_v, [hg * 128 + lanes])
        sd = plsc.load_gather(sm_v, [hg * 128 + 64 + lanes])
        sv = sa + sd
        sv = jnp.maximum(sv, 0.2 * sv)
        hoff = hg * _NP
        base_blk = s * blocks_per_tile
        if edge_split:
            base_blk = base_blk + c * (_NS * blocks_per_tile)

        def _sb(sb, carry):
            pltpu.sync_copy(src_hbm.at[pl.ds(base_blk + sb * _SB, _SB)],
                            src_v)
            pltpu.sync_copy(dst_hbm.at[pl.ds(base_blk + sb * _SB, _SB)],
                            dst_v)

            def _blk(ib, carry2):
                for g in range(_EB // 16):
                    i_s = src_v[ib, pl.ds(g * 16, 16)]
                    i_d = dst_v[ib, pl.ds(g * 16, 16)]
                    a = plsc.load_gather(as_v, [i_s])
                    b = plsc.load_gather(ad_v, [i_d])
                    z = a + b
                    zl = jnp.maximum(z, 0.2 * z)
                    w_v[pl.ds(g * 16, 16)] = jnp.exp(zl - sv)
                    gi_v[0, pl.ds(g * 16, 16)] = i_s + hoff
                    si_v[0, pl.ds(g * 16, 16)] = i_d
                pltpu.async_copy(htab_hbm.at[gi_v.at[0]], rows_v, sem).wait()

                def _scale(b2, carry3):
                    wv = plsc.load_gather(
                        w_v, [jnp.full((16,), b2, jnp.int32)])
                    for cc in range(_CH // 16):
                        rows_v[b2, pl.ds(cc * 16, 16)] = (
                            rows_v[b2, pl.ds(cc * 16, 16)] * wv)
                    return carry3
                lax.fori_loop(0, _EB, _scale, 0)
                pltpu.sync_copy(rows_v, acc_s.at[si_v.at[0]], add=True)
                return carry2
            lax.fori_loop(0, _SB, _blk, 0)
            return carry
        lax.fori_loop(0, nsb, _sb, 0)

        plsc.subcore_barrier()
        pltpu.sync_copy(
            acc_s.at[pl.ds(s * rows_per_tile, rows_per_tile)],
            out_hbm.at[pl.ds(c * _NP + s * rows_per_tile, rows_per_tile)])

    return sck


_sc_cache = {}


def _get_sc(key):
    if key not in _sc_cache:
        if key == "l1a":
            _sc_cache[key] = _make_sc_layer(
                head_offset=0, blocks_per_tile=_EPB // _NS, edge_split=False)
        elif key == "l1b":
            _sc_cache[key] = _make_sc_layer(
                head_offset=2, blocks_per_tile=_EPB // _NS, edge_split=False)
        else:
            _sc_cache[key] = _make_sc_layer(
                head_offset=0, blocks_per_tile=_EPB // (_NS * _NC),
                edge_split=True)
    return _sc_cache[key]


def _prep_edges(ei):
    loop = jnp.arange(_N, dtype=jnp.int32)
    padv = jnp.full((_EPAD - _ET,), _N, jnp.int32)
    src = jnp.concatenate([ei[0], loop, padv]).reshape(_EPB, _EB)
    dst = jnp.concatenate([ei[1], loop, padv]).reshape(_EPB, _EB)
    return src, dst


def _embed(x, ei, batch, W1, AsT1, AdT1, b1r, W2, AsT2, AdT2, b2r):
    xp = jnp.pad(x, ((0, _NP - _N), (0, 0)))
    srcB, dstB = _prep_edges(ei)
    htab, asT, adT, smax = _tc1(xp, W1, AsT1, AdT1)
    htab_f = htab.reshape(_HEADS * _NP, _CH)
    acc1a = _get_sc("l1a")(srcB, dstB, asT, adT, smax.reshape(-1), htab_f)
    acc1b = _get_sc("l1b")(srcB, dstB, asT, adT, smax.reshape(-1), htab_f)
    htab2, asT2, adT2, smax2 = _tc2(acc1a.reshape(2, _NP, _CH),
                                    acc1b.reshape(2, _NP, _CH),
                                    b1r, W2, AsT2, AdT2)
    acc2 = _get_sc("l2")(srcB, dstB, asT2, adT2, smax2.reshape(-1), htab2)
    batch3 = jnp.concatenate(
        [batch, jnp.full((_NP - _N,), _G, jnp.int32)]).reshape(_NBN, 1, _NB)
    po = _tc3(acc2.reshape(2, _NP, _CH), b2r, batch3)
    return po[:_G, :_H]


def _blockdiag_t(att, heads):
    out = jnp.zeros((8, heads * _H), jnp.float32)
    for h in range(heads):
        out = out.at[h, h * _H:(h + 1) * _H].set(att[0, h])
    return out


def kernel(x1, edge_index1, batch1, x2, edge_index2, batch2,
           W1, att_src1, att_dst1, b1, W2, att_src2, att_dst2, b2):
    AsT1 = _blockdiag_t(att_src1, _HEADS)
    AdT1 = _blockdiag_t(att_dst1, _HEADS)
    AsT2 = _blockdiag_t(att_src2, 1)
    AdT2 = _blockdiag_t(att_dst2, 1)
    b1r = b1.reshape(1, _HEADS * _H)
    b2r = b2.reshape(1, _H)
    emb1 = _embed(x1, edge_index1, batch1, W1, AsT1, AdT1, b1r,
                  W2, AsT2, AdT2, b2r)
    emb2 = _embed(x2, edge_index2, batch2, W1, AsT1, AdT1, b1r,
                  W2, AsT2, AdT2, b2r)
    return (emb1, emb2)


# speedup vs baseline: 13.3310x; 1.1435x over previous
import functools

import jax
import jax.numpy as jnp
from jax import lax
from jax.experimental import pallas as pl
from jax.experimental.pallas import tpu as pltpu
from jax.experimental.pallas import tpu_sc as plsc

_N = 10000
_D = 128
_H = 64
_HEADS = 4
_G = 16

_NP = 10240
_NB = 512
_NBN = _NP // _NB
_CH = 80
_EB = 128
_E = 320000
_ET = _E + _N
_EPB = 2816
_EPAD = _EPB * _EB
_NEG = -1e30

_NC = 2
_NS = 16


def _elu(v):
    return jnp.where(v > 0, v, jnp.exp(jnp.minimum(v, 0.0)) - 1.0)


def _tc1_body(x_ref, w_ref, ast_ref, adt_ref, htab_ref, as_ref, ad_ref, sm_ref):
    i = pl.program_id(0)
    h = jnp.dot(x_ref[...], w_ref[...], preferred_element_type=jnp.float32)
    asT = lax.dot_general(ast_ref[...], h, (((1,), (1,)), ((), ())),
                          preferred_element_type=jnp.float32)
    adT = lax.dot_general(adt_ref[...], h, (((1,), (1,)), ((), ())),
                          preferred_element_type=jnp.float32)
    gidx = i * _NB + lax.broadcasted_iota(jnp.int32, (8, _NB), 1)
    valid = gidx < _N
    asT = jnp.where(valid, asT, _NEG)
    adT = jnp.where(valid, adT, 0.0)
    as_ref[...] = asT
    ad_ref[...] = adT
    tail = (lax.broadcasted_iota(jnp.int32, (_NB, _CH - _H), 1) == 0)
    tail = tail.astype(jnp.float32)
    for hd in range(_HEADS):
        htab_ref[hd, :, 0:_H] = h[:, hd * _H:(hd + 1) * _H]
        htab_ref[hd, :, _H:_CH] = tail
    sa = jnp.max(asT, axis=1)
    sd = jnp.max(adT, axis=1)
    cur = jnp.concatenate([jnp.broadcast_to(sa[:, None], (8, 64)),
                           jnp.broadcast_to(sd[:, None], (8, 64))], axis=1)

    @pl.when(i == 0)
    def _():
        sm_ref[...] = cur

    @pl.when(i > 0)
    def _():
        sm_ref[...] = jnp.maximum(sm_ref[...], cur)


_tc1 = pl.pallas_call(
    _tc1_body,
    grid=(_NBN,),
    in_specs=[
        pl.BlockSpec((_NB, _D), lambda i: (i, 0)),
        pl.BlockSpec((_D, _HEADS * _H), lambda i: (0, 0)),
        pl.BlockSpec((8, _HEADS * _H), lambda i: (0, 0)),
        pl.BlockSpec((8, _HEADS * _H), lambda i: (0, 0)),
    ],
    out_specs=[
        pl.BlockSpec((_HEADS, _NB, _CH), lambda i: (0, i, 0)),
        pl.BlockSpec((8, _NB), lambda i: (0, i)),
        pl.BlockSpec((8, _NB), lambda i: (0, i)),
        pl.BlockSpec((8, 128), lambda i: (0, 0)),
    ],
    out_shape=[
        jax.ShapeDtypeStruct((_HEADS, _NP, _CH), jnp.float32),
        jax.ShapeDtypeStruct((8, _NP), jnp.float32),
        jax.ShapeDtypeStruct((8, _NP), jnp.float32),
        jax.ShapeDtypeStruct((8, 128), jnp.float32),
    ],
)


def _tc2_body(acca_ref, accb_ref, b1_ref, w2_ref, ast_ref, adt_ref,
              htab_ref, as_ref, ad_ref, sm_ref):
    i = pl.program_id(0)
    xs = []
    for hd in range(_HEADS):
        blk = acca_ref[hd] if hd < 2 else accb_ref[hd - 2]
        num = blk[:, 0:_H]
        den = blk[:, _H:_H + 1]
        v = num / (den + 1e-16) + b1_ref[0:1, hd * _H:(hd + 1) * _H]
        xs.append(_elu(v))
    x2 = jnp.concatenate(xs, axis=1)
    h2 = jnp.dot(x2, w2_ref[...], preferred_element_type=jnp.float32)
    asT = lax.dot_general(ast_ref[...], h2, (((1,), (1,)), ((), ())),
                          preferred_element_type=jnp.float32)
    adT = lax.dot_general(adt_ref[...], h2, (((1,), (1,)), ((), ())),
                          preferred_element_type=jnp.float32)
    gidx = i * _NB + lax.broadcasted_iota(jnp.int32, (8, _NB), 1)
    valid = gidx < _N
    asT = jnp.where(valid, asT, _NEG)
    adT = jnp.where(valid, adT, 0.0)
    as_ref[...] = asT
    ad_ref[...] = adT
    tail = (lax.broadcasted_iota(jnp.int32, (_NB, _CH - _H), 1) == 0)
    htab_ref[:, 0:_H] = h2
    htab_ref[:, _H:_CH] = tail.astype(jnp.float32)
    sa = jnp.max(asT, axis=1)
    sd = jnp.max(adT, axis=1)
    cur = jnp.concatenate([jnp.broadcast_to(sa[:, None], (8, 64)),
                           jnp.broadcast_to(sd[:, None], (8, 64))], axis=1)

    @pl.when(i == 0)
    def _():
        sm_ref[...] = cur

    @pl.when(i > 0)
    def _():
        sm_ref[...] = jnp.maximum(sm_ref[...], cur)


_tc2 = pl.pallas_call(
    _tc2_body,
    grid=(_NBN,),
    in_specs=[
        pl.BlockSpec((2, _NB, _CH), lambda i: (0, i, 0)),
        pl.BlockSpec((2, _NB, _CH), lambda i: (0, i, 0)),
        pl.BlockSpec((1, _HEADS * _H), lambda i: (0, 0)),
        pl.BlockSpec((_HEADS * _H, _H), lambda i: (0, 0)),
        pl.BlockSpec((8, _H), lambda i: (0, 0)),
        pl.BlockSpec((8, _H), lambda i: (0, 0)),
    ],
    out_specs=[
        pl.BlockSpec((_NB, _CH), lambda i: (i, 0)),
        pl.BlockSpec((8, _NB), lambda i: (0, i)),
        pl.BlockSpec((8, _NB), lambda i: (0, i)),
        pl.BlockSpec((8, 128), lambda i: (0, 0)),
    ],
    out_shape=[
        jax.ShapeDtypeStruct((_NP, _CH), jnp.float32),
        jax.ShapeDtypeStruct((8, _NP), jnp.float32),
        jax.ShapeDtypeStruct((8, _NP), jnp.float32),
        jax.ShapeDtypeStruct((8, 128), jnp.float32),
    ],
)


def _tc3_body(acc_ref, b2_ref, batch_ref, po_ref):
    i = pl.program_id(0)
    num = acc_ref[0, :, 0:_H] + acc_ref[1, :, 0:_H]
    den = acc_ref[0, :, _H:_H + 1] + acc_ref[1, :, _H:_H + 1]
    o = _elu(num / (den + 1e-16) + b2_ref[0:1, :])
    tail = (lax.broadcasted_iota(jnp.int32, (_NB, 64), 1) == 0)
    oa = jnp.concatenate([o, tail.astype(jnp.float32)], axis=1)
    bt = batch_ref[0, 0, :]
    P = (lax.broadcasted_iota(jnp.int32, (_G, _NB), 0)
         == bt[None, :]).astype(jnp.float32)
    contrib = jnp.dot(P, oa, preferred_element_type=jnp.float32)

    @pl.when(i == 0)
    def _():
        po_ref[...] = contrib

    @pl.when(i > 0)
    def _():
        po_ref[...] = po_ref[...] + contrib

    @pl.when(i == _NBN - 1)
    def _():
        s = po_ref[...]
        cnt = jnp.maximum(s[:, _H:_H + 1], 1.0)
        po_ref[...] = s / cnt


_tc3 = pl.pallas_call(
    _tc3_body,
    grid=(_NBN,),
    in_specs=[
        pl.BlockSpec((2, _NB, _CH), lambda i: (0, i, 0)),
        pl.BlockSpec((1, _H), lambda i: (0, 0)),
        pl.BlockSpec((1, 1, _NB), lambda i: (i, 0, 0)),
    ],
    out_specs=pl.BlockSpec((_G, 128), lambda i: (0, 0)),
    out_shape=jax.ShapeDtypeStruct((_G, 128), jnp.float32),
)


_KP = 4


def _make_sc_layer(head_offset, blocks_per_tile, edge_split):
    rows_per_tile = _NP // _NS
    ngrp = blocks_per_tile // _KP
    mesh = plsc.VectorSubcoreMesh(core_axis_name="c", subcore_axis_name="s",
                                  num_cores=_NC, num_subcores=_NS)

    @functools.partial(
        pl.kernel,
        out_type=jax.ShapeDtypeStruct((2 * _NP, _CH), jnp.float32),
        mesh=mesh,
        compiler_params=pltpu.CompilerParams(needs_layout_passes=False,
                                             use_tc_tiling_on_sc=False),
        scratch_types=[
            pltpu.VMEM((2 * _KP, _EB), jnp.int32),
            pltpu.VMEM((2 * _KP, _EB), jnp.int32),
            pltpu.VMEM((_NP,), jnp.float32),
            pltpu.VMEM((_NP,), jnp.float32),
            pltpu.VMEM((1024,), jnp.float32),
            pltpu.VMEM((_KP * _EB, _CH), jnp.float32),
            pltpu.VMEM((_KP * _EB,), jnp.float32),
            pltpu.VMEM((_KP, _EB), jnp.int32),
            pltpu.VMEM((_KP, _EB), jnp.int32),
            pltpu.VMEM_SHARED((_NP, _CH), jnp.float32),
            pltpu.SemaphoreType.DMA,
            pltpu.SemaphoreType.DMA,
            pltpu.SemaphoreType.DMA,
        ],
    )
    def sck(src_hbm, dst_hbm, ast_hbm, adt_hbm, sm_hbm, htab_hbm, out_hbm,
            src_v, dst_v, as_v, ad_v, sm_v, rows_v, w_v, gi_v, si_v,
            acc_s, sem_g, sem_i, sem_s):
        c = lax.axis_index("c")
        s = lax.axis_index("s")
        hg = head_offset if edge_split else head_offset + c

        def _zb(b, carry):
            for cc in range(_CH // 16):
                rows_v[b, pl.ds(cc * 16, 16)] = jnp.zeros((16,), jnp.float32)
            return carry
        lax.fori_loop(0, _EB, _zb, 0)
        for z in range(rows_per_tile // _EB):
            pltpu.sync_copy(
                rows_v.at[pl.ds(0, _EB)],
                acc_s.at[pl.ds(s * rows_per_tile + z * _EB, _EB)])

        pltpu.sync_copy(ast_hbm.at[hg], as_v)
        pltpu.sync_copy(adt_hbm.at[hg], ad_v)
        pltpu.sync_copy(sm_hbm, sm_v)
        plsc.subcore_barrier()

        lanes = lax.iota(jnp.int32, 16)
        sa = plsc.load_gather(sm_v, [hg * 128 + lanes])
        sd = plsc.load_gather(sm_v, [hg * 128 + 64 + lanes])
        sv = sa + sd
        sv = jnp.maximum(sv, 0.2 * sv)
        hoff = hg * _NP
        base_blk = s * blocks_per_tile
        if edge_split:
            base_blk = base_blk + c * (_NS * blocks_per_tile)

        pltpu.sync_copy(src_hbm.at[pl.ds(base_blk, _KP)],
                        src_v.at[pl.ds(0, _KP)])
        pltpu.sync_copy(dst_hbm.at[pl.ds(base_blk, _KP)],
                        dst_v.at[pl.ds(0, _KP)])

        def _grp(g, carry):
            po = lax.rem(g, 2) * _KP
            pn = lax.rem(g + 1, 2) * _KP

            @pl.when(g > 0)
            def _():
                pltpu.make_async_copy(
                    src_hbm.at[pl.ds(base_blk, _KP)],
                    src_v.at[pl.ds(po, _KP)], sem_i).wait()
                pltpu.make_async_copy(
                    dst_hbm.at[pl.ds(base_blk, _KP)],
                    dst_v.at[pl.ds(po, _KP)], sem_i).wait()

            @pl.when(g + 1 < ngrp)
            def _():
                nb = base_blk + (g + 1) * _KP
                pltpu.async_copy(src_hbm.at[pl.ds(nb, _KP)],
                                 src_v.at[pl.ds(pn, _KP)], sem_i)
                pltpu.async_copy(dst_hbm.at[pl.ds(nb, _KP)],
                                 dst_v.at[pl.ds(pn, _KP)], sem_i)

            gd = []
            for k in range(_KP):
                for gg in range(_EB // 16):
                    i_s = src_v[po + k, pl.ds(gg * 16, 16)]
                    i_d = dst_v[po + k, pl.ds(gg * 16, 16)]
                    a = plsc.load_gather(as_v, [i_s])
                    b = plsc.load_gather(ad_v, [i_d])
                    z = a + b
                    zl = jnp.maximum(z, 0.2 * z)
                    w_v[pl.ds(k * _EB + gg * 16, 16)] = jnp.exp(zl - sv)
                    gi_v[k, pl.ds(gg * 16, 16)] = i_s + hoff
                    si_v[k, pl.ds(gg * 16, 16)] = i_d
                gd.append(pltpu.async_copy(
                    htab_hbm.at[gi_v.at[k]],
                    rows_v.at[pl.ds(k * _EB, _EB)], sem_g))

            sdl = []
            for k in range(_KP):
                gd[k].wait()

                def _scale(b2, carry3):
                    wv = plsc.load_gather(
                        w_v, [jnp.full((16,), b2, jnp.int32)])
                    for cc in range(_CH // 16):
                        rows_v[b2, pl.ds(cc * 16, 16)] = (
                            rows_v[b2, pl.ds(cc * 16, 16)] * wv)
                    return carry3
                lax.fori_loop(k * _EB, (k + 1) * _EB, _scale, 0)
                sdl.append(pltpu.async_copy(
                    rows_v.at[pl.ds(k * _EB, _EB)],
                    acc_s.at[si_v.at[k]], sem_s, add=True))
            for k in range(_KP):
                sdl[k].wait()
            return carry
        lax.fori_loop(0, ngrp, _grp, 0)

        plsc.subcore_barrier()
        pltpu.sync_copy(
            acc_s.at[pl.ds(s * rows_per_tile, rows_per_tile)],
            out_hbm.at[pl.ds(c * _NP + s * rows_per_tile, rows_per_tile)])

    return sck


_sc_cache = {}


def _get_sc(key):
    if key not in _sc_cache:
        if key == "l1a":
            _sc_cache[key] = _make_sc_layer(
                head_offset=0, blocks_per_tile=_EPB // _NS, edge_split=False)
        elif key == "l1b":
            _sc_cache[key] = _make_sc_layer(
                head_offset=2, blocks_per_tile=_EPB // _NS, edge_split=False)
        else:
            _sc_cache[key] = _make_sc_layer(
                head_offset=0, blocks_per_tile=_EPB // (_NS * _NC),
                edge_split=True)
    return _sc_cache[key]


def _prep_edges(ei):
    loop = jnp.arange(_N, dtype=jnp.int32)
    padv = jnp.full((_EPAD - _ET,), _N, jnp.int32)
    src = jnp.concatenate([ei[0], loop, padv]).reshape(_EPB, _EB)
    dst = jnp.concatenate([ei[1], loop, padv]).reshape(_EPB, _EB)
    return src, dst


def _embed(x, ei, batch, W1, AsT1, AdT1, b1r, W2, AsT2, AdT2, b2r):
    xp = jnp.pad(x, ((0, _NP - _N), (0, 0)))
    srcB, dstB = _prep_edges(ei)
    htab, asT, adT, smax = _tc1(xp, W1, AsT1, AdT1)
    htab_f = htab.reshape(_HEADS * _NP, _CH)
    acc1a = _get_sc("l1a")(srcB, dstB, asT, adT, smax.reshape(-1), htab_f)
    acc1b = _get_sc("l1b")(srcB, dstB, asT, adT, smax.reshape(-1), htab_f)
    htab2, asT2, adT2, smax2 = _tc2(acc1a.reshape(2, _NP, _CH),
                                    acc1b.reshape(2, _NP, _CH),
                                    b1r, W2, AsT2, AdT2)
    acc2 = _get_sc("l2")(srcB, dstB, asT2, adT2, smax2.reshape(-1), htab2)
    batch3 = jnp.concatenate(
        [batch, jnp.full((_NP - _N,), _G, jnp.int32)]).reshape(_NBN, 1, _NB)
    po = _tc3(acc2.reshape(2, _NP, _CH), b2r, batch3)
    return po[:_G, :_H]


def _blockdiag_t(att, heads):
    out = jnp.zeros((8, heads * _H), jnp.float32)
    for h in range(heads):
        out = out.at[h, h * _H:(h + 1) * _H].set(att[0, h])
    return out


def kernel(x1, edge_index1, batch1, x2, edge_index2, batch2,
           W1, att_src1, att_dst1, b1, W2, att_src2, att_dst2, b2):
    AsT1 = _blockdiag_t(att_src1, _HEADS)
    AdT1 = _blockdiag_t(att_dst1, _HEADS)
    AsT2 = _blockdiag_t(att_src2, 1)
    AdT2 = _blockdiag_t(att_dst2, 1)
    b1r = b1.reshape(1, _HEADS * _H)
    b2r = b2.reshape(1, _H)
    emb1 = _embed(x1, edge_index1, batch1, W1, AsT1, AdT1, b1r,
                  W2, AsT2, AdT2, b2r)
    emb2 = _embed(x2, edge_index2, batch2, W1, AsT1, AdT1, b1r,
                  W2, AsT2, AdT2, b2r)
    return (emb1, emb2)


# speedup vs baseline: 33.4682x; 2.5106x over previous
import functools

import jax
import jax.numpy as jnp
import numpy as np
from jax import lax
from jax.experimental import pallas as pl
from jax.experimental.pallas import tpu as pltpu
from jax.experimental.pallas import tpu_sc as plsc

_N = 10000
_D = 128
_H = 64
_HEADS = 4
_G = 16

_NP = 10240
_NB = 512
_NBN = _NP // _NB
_EB = 128
_E = 320000
_ET = _E + _N
_EPB = 2816
_EPAD = _EPB * _EB
_NEG = -1e30

_NC = 2
_NS = 16
_KP = 4

_PERM = np.zeros(_H, np.int32)
for _m in range(_H):
    _q, _r = divmod(_m, 32)
    _PERM[_m] = 32 * _q + (_r // 2 if _r % 2 == 0 else 16 + _r // 2)


def _elu(v):
    return jnp.where(v > 0, v, jnp.exp(jnp.minimum(v, 0.0)) - 1.0)


def _tc1_body(x_ref, w_ref, ast_ref, adt_ref, htab_ref, as_ref, ad_ref, sm_ref):
    i = pl.program_id(0)
    h = jnp.dot(x_ref[...], w_ref[...], preferred_element_type=jnp.float32)
    asT = lax.dot_general(ast_ref[...], h, (((1,), (1,)), ((), ())),
                          preferred_element_type=jnp.float32)
    adT = lax.dot_general(adt_ref[...], h, (((1,), (1,)), ((), ())),
                          preferred_element_type=jnp.float32)
    gidx = i * _NB + lax.broadcasted_iota(jnp.int32, (8, _NB), 1)
    valid = gidx < _N
    asT = jnp.where(valid, asT, _NEG)
    adT = jnp.where(valid, adT, 0.0)
    as_ref[...] = asT
    ad_ref[...] = adT
    for hd in range(_HEADS):
        htab_ref[hd, :, :] = h[:, hd * _H:(hd + 1) * _H]
    sa = jnp.max(asT, axis=1)
    sd = jnp.max(adT, axis=1)
    cur = jnp.concatenate([jnp.broadcast_to(sa[:, None], (8, 64)),
                           jnp.broadcast_to(sd[:, None], (8, 64))], axis=1)

    @pl.when(i == 0)
    def _():
        sm_ref[...] = cur

    @pl.when(i > 0)
    def _():
        sm_ref[...] = jnp.maximum(sm_ref[...], cur)


_tc1 = pl.pallas_call(
    _tc1_body,
    grid=(_NBN,),
    in_specs=[
        pl.BlockSpec((_NB, _D), lambda i: (i, 0)),
        pl.BlockSpec((_D, _HEADS * _H), lambda i: (0, 0)),
        pl.BlockSpec((8, _HEADS * _H), lambda i: (0, 0)),
        pl.BlockSpec((8, _HEADS * _H), lambda i: (0, 0)),
    ],
    out_specs=[
        pl.BlockSpec((_HEADS, _NB, _H), lambda i: (0, i, 0)),
        pl.BlockSpec((8, _NB), lambda i: (0, i)),
        pl.BlockSpec((8, _NB), lambda i: (0, i)),
        pl.BlockSpec((8, 128), lambda i: (0, 0)),
    ],
    out_shape=[
        jax.ShapeDtypeStruct((_HEADS, _NP, _H), jnp.float32),
        jax.ShapeDtypeStruct((8, _NP), jnp.float32),
        jax.ShapeDtypeStruct((8, _NP), jnp.float32),
        jax.ShapeDtypeStruct((8, 128), jnp.float32),
    ],
)


def _tc2_body(fa_ref, da_ref, fb_ref, db_ref, b1_ref, w2_ref, ast_ref, adt_ref,
              htab_ref, as_ref, ad_ref, sm_ref):
    i = pl.program_id(0)
    xs = []
    for hd in range(_HEADS):
        f_ref, d_ref = (fa_ref, da_ref) if hd < 2 else (fb_ref, db_ref)
        num = f_ref[hd % 2]
        den = d_ref[hd % 2][:, 0:1]
        v = num / (den + 1e-16) + b1_ref[0:1, hd * _H:(hd + 1) * _H]
        xs.append(_elu(v))
    x2 = jnp.concatenate(xs, axis=1)
    h2 = jnp.dot(x2, w2_ref[...], preferred_element_type=jnp.float32)
    asT = lax.dot_general(ast_ref[...], h2, (((1,), (1,)), ((), ())),
                          preferred_element_type=jnp.float32)
    adT = lax.dot_general(adt_ref[...], h2, (((1,), (1,)), ((), ())),
                          preferred_element_type=jnp.float32)
    gidx = i * _NB + lax.broadcasted_iota(jnp.int32, (8, _NB), 1)
    valid = gidx < _N
    asT = jnp.where(valid, asT, _NEG)
    adT = jnp.where(valid, adT, 0.0)
    as_ref[...] = asT
    ad_ref[...] = adT
    htab_ref[...] = h2
    sa = jnp.max(asT, axis=1)
    sd = jnp.max(adT, axis=1)
    cur = jnp.concatenate([jnp.broadcast_to(sa[:, None], (8, 64)),
                           jnp.broadcast_to(sd[:, None], (8, 64))], axis=1)

    @pl.when(i == 0)
    def _():
        sm_ref[...] = cur

    @pl.when(i > 0)
    def _():
        sm_ref[...] = jnp.maximum(sm_ref[...], cur)


_tc2 = pl.pallas_call(
    _tc2_body,
    grid=(_NBN,),
    in_specs=[
        pl.BlockSpec((2, _NB, _H), lambda i: (0, i, 0)),
        pl.BlockSpec((2, _NB, 16), lambda i: (0, i, 0)),
        pl.BlockSpec((2, _NB, _H), lambda i: (0, i, 0)),
        pl.BlockSpec((2, _NB, 16), lambda i: (0, i, 0)),
        pl.BlockSpec((1, _HEADS * _H), lambda i: (0, 0)),
        pl.BlockSpec((_HEADS * _H, _H), lambda i: (0, 0)),
        pl.BlockSpec((8, _H), lambda i: (0, 0)),
        pl.BlockSpec((8, _H), lambda i: (0, 0)),
    ],
    out_specs=[
        pl.BlockSpec((_NB, _H), lambda i: (i, 0)),
        pl.BlockSpec((8, _NB), lambda i: (0, i)),
        pl.BlockSpec((8, _NB), lambda i: (0, i)),
        pl.BlockSpec((8, 128), lambda i: (0, 0)),
    ],
    out_shape=[
        jax.ShapeDtypeStruct((_NP, _H), jnp.float32),
        jax.ShapeDtypeStruct((8, _NP), jnp.float32),
        jax.ShapeDtypeStruct((8, _NP), jnp.float32),
        jax.ShapeDtypeStruct((8, 128), jnp.float32),
    ],
)


def _tc3_body(f_ref, d_ref, b2_ref, batch_ref, po_ref):
    i = pl.program_id(0)
    num = f_ref[0] + f_ref[1]
    den = d_ref[0][:, 0:1] + d_ref[1][:, 0:1]
    o = _elu(num / (den + 1e-16) + b2_ref[0:1, :])
    tail = (lax.broadcasted_iota(jnp.int32, (_NB, 64), 1) == 0)
    oa = jnp.concatenate([o, tail.astype(jnp.float32)], axis=1)
    bt = batch_ref[0, 0, :]
    P = (lax.broadcasted_iota(jnp.int32, (_G, _NB), 0)
         == bt[None, :]).astype(jnp.float32)
    contrib = jnp.dot(P, oa, preferred_element_type=jnp.float32)

    @pl.when(i == 0)
    def _():
        po_ref[...] = contrib

    @pl.when(i > 0)
    def _():
        po_ref[...] = po_ref[...] + contrib

    @pl.when(i == _NBN - 1)
    def _():
        s = po_ref[...]
        cnt = jnp.maximum(s[:, _H:_H + 1], 1.0)
        po_ref[...] = s / cnt


_tc3 = pl.pallas_call(
    _tc3_body,
    grid=(_NBN,),
    in_specs=[
        pl.BlockSpec((2, _NB, _H), lambda i: (0, i, 0)),
        pl.BlockSpec((2, _NB, 16), lambda i: (0, i, 0)),
        pl.BlockSpec((1, _H), lambda i: (0, 0)),
        pl.BlockSpec((1, 1, _NB), lambda i: (i, 0, 0)),
    ],
    out_specs=pl.BlockSpec((_G, 128), lambda i: (0, 0)),
    out_shape=jax.ShapeDtypeStruct((_G, 128), jnp.float32),
)


def _make_sc_layer(head_offset, blocks_per_tile, edge_split):
    rows_per_tile = _NP // _NS
    ngrp = blocks_per_tile // _KP
    mesh = plsc.VectorSubcoreMesh(core_axis_name="c", subcore_axis_name="s",
                                  num_cores=_NC, num_subcores=_NS)

    @functools.partial(
        pl.kernel,
        out_type=[jax.ShapeDtypeStruct((2 * _NP, _H), jnp.float32),
                  jax.ShapeDtypeStruct((2 * _NP, 16), jnp.float32)],
        mesh=mesh,
        compiler_params=pltpu.CompilerParams(needs_layout_passes=False,
                                             use_tc_tiling_on_sc=False),
        scratch_types=[
            pltpu.VMEM((2 * _KP, _EB), jnp.int32),
            pltpu.VMEM((2 * _KP, _EB), jnp.int32),
            pltpu.VMEM((_NP,), jnp.float32),
            pltpu.VMEM((_NP,), jnp.float32),
            pltpu.VMEM((1024,), jnp.float32),
            pltpu.VMEM((_KP * _EB, _H), jnp.bfloat16),
            pltpu.VMEM((2 * _EB, _H), jnp.float32),
            pltpu.VMEM((_KP * _EB, 16), jnp.float32),
            pltpu.VMEM((_KP * _EB,), jnp.float32),
            pltpu.VMEM((_KP, _EB), jnp.int32),
            pltpu.VMEM((_KP, _EB), jnp.int32),
            pltpu.VMEM_SHARED((_NP, _H), jnp.float32),
            pltpu.VMEM_SHARED((_NP, 16), jnp.float32),
            pltpu.SemaphoreType.DMA,
            pltpu.SemaphoreType.DMA,
            pltpu.SemaphoreType.DMA,
        ],
    )
    def sck(src_hbm, dst_hbm, ast_hbm, adt_hbm, sm_hbm, htab_hbm,
            of_hbm, od_hbm,
            src_v, dst_v, as_v, ad_v, sm_v, rows_v, frows_v, den_v, w_v,
            gi_v, si_v, accf_s, accd_s, sem_g, sem_i, sem_s):
        c = lax.axis_index("c")
        s = lax.axis_index("s")
        hg = head_offset if edge_split else head_offset + c
        lanes = lax.iota(jnp.int32, 16)
        zeros16 = jnp.zeros((16,), jnp.float32)
        izeros16 = jnp.zeros((16,), jnp.int32)

        def _zf(b, carry):
            for cc in range(_H // 16):
                frows_v[b, pl.ds(cc * 16, 16)] = zeros16
            return carry
        lax.fori_loop(0, 2 * _EB, _zf, 0)

        def _zd(b, carry):
            den_v[b, pl.ds(0, 16)] = zeros16
            return carry
        lax.fori_loop(0, _KP * _EB, _zd, 0)
        for z in range(rows_per_tile // _EB):
            pltpu.sync_copy(
                frows_v.at[pl.ds(0, _EB)],
                accf_s.at[pl.ds(s * rows_per_tile + z * _EB, _EB)])
        pltpu.sync_copy(den_v.at[pl.ds(0, _KP * _EB)],
                        accd_s.at[pl.ds(s * rows_per_tile, _KP * _EB)])
        pltpu.sync_copy(den_v.at[pl.ds(0, rows_per_tile - _KP * _EB)],
                        accd_s.at[pl.ds(s * rows_per_tile + _KP * _EB,
                                        rows_per_tile - _KP * _EB)])

        pltpu.sync_copy(ast_hbm.at[hg], as_v)
        pltpu.sync_copy(adt_hbm.at[hg], ad_v)
        pltpu.sync_copy(sm_hbm, sm_v)
        plsc.subcore_barrier()

        sa = plsc.load_gather(sm_v, [hg * 128 + lanes])
        sd = plsc.load_gather(sm_v, [hg * 128 + 64 + lanes])
        sv = sa + sd
        sv = jnp.maximum(sv, 0.2 * sv)
        hoff = hg * _NP
        base_blk = s * blocks_per_tile
        if edge_split:
            base_blk = base_blk + c * (_NS * blocks_per_tile)

        pltpu.sync_copy(src_hbm.at[pl.ds(base_blk, _KP)],
                        src_v.at[pl.ds(0, _KP)])
        pltpu.sync_copy(dst_hbm.at[pl.ds(base_blk, _KP)],
                        dst_v.at[pl.ds(0, _KP)])

        def _grp(g, carry):
            po = lax.rem(g, 2) * _KP
            pn = lax.rem(g + 1, 2) * _KP

            @pl.when(g > 0)
            def _():
                pltpu.make_async_copy(
                    src_hbm.at[pl.ds(base_blk, _KP)],
                    src_v.at[pl.ds(po, _KP)], sem_i).wait()
                pltpu.make_async_copy(
                    dst_hbm.at[pl.ds(base_blk, _KP)],
                    dst_v.at[pl.ds(po, _KP)], sem_i).wait()

            @pl.when(g + 1 < ngrp)
            def _():
                nb = base_blk + (g + 1) * _KP
                pltpu.async_copy(src_hbm.at[pl.ds(nb, _KP)],
                                 src_v.at[pl.ds(pn, _KP)], sem_i)
                pltpu.async_copy(dst_hbm.at[pl.ds(nb, _KP)],
                                 dst_v.at[pl.ds(pn, _KP)], sem_i)

            gd = []
            for k in range(_KP):
                for gg in range(_EB // 16):
                    i_s = src_v[po + k, pl.ds(gg * 16, 16)]
                    i_d = dst_v[po + k, pl.ds(gg * 16, 16)]
                    a = plsc.load_gather(as_v, [i_s])
                    b = plsc.load_gather(ad_v, [i_d])
                    z = a + b
                    zl = jnp.maximum(z, 0.2 * z)
                    w = jnp.exp(zl - sv)
                    w_v[pl.ds(k * _EB + gg * 16, 16)] = w
                    plsc.store_scatter(
                        den_v, [k * _EB + gg * 16 + lanes, izeros16], w)
                    gi_v[k, pl.ds(gg * 16, 16)] = i_s + hoff
                    si_v[k, pl.ds(gg * 16, 16)] = i_d
                gd.append(pltpu.async_copy(
                    htab_hbm.at[gi_v.at[k]],
                    rows_v.at[pl.ds(k * _EB, _EB)], sem_g))

            sdl = []
            for k in range(_KP):
                gd[k].wait()
                if k >= 2:
                    sdl[2 * (k - 2)].wait()
                fbase = (k % 2) * _EB

                def _scale2(b2, carry3):
                    wv = plsc.load_gather(
                        w_v, [jnp.full((16,), b2, jnp.int32)])
                    fb = fbase + b2 - k * _EB
                    for cc in range(_H // 32):
                        t = rows_v[b2, pl.ds(cc * 32, 32)]
                        ev, od = plsc.unpack(
                            t, format=plsc.PackFormat.INTERLEAVED)
                        frows_v[fb, pl.ds(cc * 32, 16)] = ev * wv
                        frows_v[fb, pl.ds(cc * 32 + 16, 16)] = od * wv
                    return carry3
                lax.fori_loop(k * _EB, (k + 1) * _EB, _scale2, 0)
                sdl.append(pltpu.async_copy(
                    frows_v.at[pl.ds(fbase, _EB)],
                    accf_s.at[si_v.at[k]], sem_s, add=True))
                sdl.append(pltpu.async_copy(
                    den_v.at[pl.ds(k * _EB, _EB)],
                    accd_s.at[si_v.at[k]], sem_s, add=True))
            for d in sdl[2 * (_KP - 2):]:
                d.wait()
            for k in range(_KP - 2):
                sdl[2 * k + 1].wait()
            return carry
        lax.fori_loop(0, ngrp, _grp, 0)

        plsc.subcore_barrier()
        pltpu.sync_copy(
            accf_s.at[pl.ds(s * rows_per_tile, rows_per_tile)],
            of_hbm.at[pl.ds(c * _NP + s * rows_per_tile, rows_per_tile)])
        pltpu.sync_copy(
            accd_s.at[pl.ds(s * rows_per_tile, rows_per_tile)],
            od_hbm.at[pl.ds(c * _NP + s * rows_per_tile, rows_per_tile)])

    return sck


_sc_cache = {}


def _get_sc(key):
    if key not in _sc_cache:
        if key == "l1a":
            _sc_cache[key] = _make_sc_layer(
                head_offset=0, blocks_per_tile=_EPB // _NS, edge_split=False)
        elif key == "l1b":
            _sc_cache[key] = _make_sc_layer(
                head_offset=2, blocks_per_tile=_EPB // _NS, edge_split=False)
        else:
            _sc_cache[key] = _make_sc_layer(
                head_offset=0, blocks_per_tile=_EPB // (_NS * _NC),
                edge_split=True)
    return _sc_cache[key]


def _prep_edges(ei):
    loop = jnp.arange(_N, dtype=jnp.int32)
    padv = jnp.full((_EPAD - _ET,), _N, jnp.int32)
    src = jnp.concatenate([ei[0], loop, padv]).reshape(_EPB, _EB)
    dst = jnp.concatenate([ei[1], loop, padv]).reshape(_EPB, _EB)
    return src, dst


def _embed(x, ei, batch, W1, AsT1, AdT1, b1r, W2, AsT2, AdT2, b2r):
    xp = jnp.pad(x, ((0, _NP - _N), (0, 0)))
    srcB, dstB = _prep_edges(ei)
    htab, asT, adT, smax = _tc1(xp, W1, AsT1, AdT1)
    htab_bf = htab[:, :, _PERM].astype(jnp.bfloat16).reshape(_HEADS * _NP, _H)
    sm = smax.reshape(-1)
    f1a, d1a = _get_sc("l1a")(srcB, dstB, asT, adT, sm, htab_bf)
    f1b, d1b = _get_sc("l1b")(srcB, dstB, asT, adT, sm, htab_bf)
    htab2, asT2, adT2, smax2 = _tc2(f1a.reshape(2, _NP, _H),
                                    d1a.reshape(2, _NP, 16),
                                    f1b.reshape(2, _NP, _H),
                                    d1b.reshape(2, _NP, 16),
                                    b1r, W2, AsT2, AdT2)
    htab2_bf = htab2[:, _PERM].astype(jnp.bfloat16)
    f2, d2 = _get_sc("l2")(srcB, dstB, asT2, adT2, smax2.reshape(-1),
                           htab2_bf)
    batch3 = jnp.concatenate(
        [batch, jnp.full((_NP - _N,), _G, jnp.int32)]).reshape(_NBN, 1, _NB)
    po = _tc3(f2.reshape(2, _NP, _H), d2.reshape(2, _NP, 16), b2r, batch3)
    return po[:_G, :_H]


def _blockdiag_t(att, heads):
    out = jnp.zeros((8, heads * _H), jnp.float32)
    for h in range(heads):
        out = out.at[h, h * _H:(h + 1) * _H].set(att[0, h])
    return out


def kernel(x1, edge_index1, batch1, x2, edge_index2, batch2,
           W1, att_src1, att_dst1, b1, W2, att_src2, att_dst2, b2):
    AsT1 = _blockdiag_t(att_src1, _HEADS)
    AdT1 = _blockdiag_t(att_dst1, _HEADS)
    AsT2 = _blockdiag_t(att_src2, 1)
    AdT2 = _blockdiag_t(att_dst2, 1)
    b1r = b1.reshape(1, _HEADS * _H)
    b2r = b2.reshape(1, _H)
    emb1 = _embed(x1, edge_index1, batch1, W1, AsT1, AdT1, b1r,
                  W2, AsT2, AdT2, b2r)
    emb2 = _embed(x2, edge_index2, batch2, W1, AsT1, AdT1, b1r,
                  W2, AsT2, AdT2, b2r)
    return (emb1, emb2)


# speedup vs baseline: 39.4016x; 1.1773x over previous
import functools

import jax
import jax.numpy as jnp
import numpy as np
from jax import lax
from jax.experimental import pallas as pl
from jax.experimental.pallas import tpu as pltpu
from jax.experimental.pallas import tpu_sc as plsc

_N = 10000
_D = 128
_H = 64
_HEADS = 4
_G = 16

_NP = 10240
_NB = 512
_NBN = _NP // _NB
_EB = 128
_E = 320000
_ET = _E + _N
_EPB1 = 2688
_EPB2 = 2816
_EPAD = _EPB2 * _EB
_NEG = -1e30

_NC = 2
_NS = 16
_KP = 4

_PERM = np.zeros(_H, np.int32)
for _m in range(_H):
    _q, _r = divmod(_m, 32)
    _PERM[_m] = 32 * _q + (_r // 2 if _r % 2 == 0 else 16 + _r // 2)


def _elu(v):
    return jnp.where(v > 0, v, jnp.exp(jnp.minimum(v, 0.0)) - 1.0)


def _tc1_body(x_ref, w_ref, ast_ref, adt_ref, htab_ref, as_ref, ad_ref, sm_ref):
    i = pl.program_id(0)
    h = jnp.dot(x_ref[...], w_ref[...], preferred_element_type=jnp.float32)
    asT = lax.dot_general(ast_ref[...], h, (((1,), (1,)), ((), ())),
                          preferred_element_type=jnp.float32)
    adT = lax.dot_general(adt_ref[...], h, (((1,), (1,)), ((), ())),
                          preferred_element_type=jnp.float32)
    gidx = i * _NB + lax.broadcasted_iota(jnp.int32, (8, _NB), 1)
    valid = gidx < _N
    asT = jnp.where(valid, asT, _NEG)
    adT = jnp.where(valid, adT, 0.0)
    as_ref[...] = asT
    ad_ref[...] = adT
    for hd in range(_HEADS):
        htab_ref[hd, :, :] = h[:, hd * _H:(hd + 1) * _H]
    sa = jnp.max(asT, axis=1)
    sd = jnp.max(adT, axis=1)
    cur = jnp.concatenate([jnp.broadcast_to(sa[:, None], (8, 64)),
                           jnp.broadcast_to(sd[:, None], (8, 64))], axis=1)

    @pl.when(i == 0)
    def _():
        sm_ref[...] = cur

    @pl.when(i > 0)
    def _():
        sm_ref[...] = jnp.maximum(sm_ref[...], cur)


_tc1 = pl.pallas_call(
    _tc1_body,
    grid=(_NBN,),
    in_specs=[
        pl.BlockSpec((_NB, _D), lambda i: (i, 0)),
        pl.BlockSpec((_D, _HEADS * _H), lambda i: (0, 0)),
        pl.BlockSpec((8, _HEADS * _H), lambda i: (0, 0)),
        pl.BlockSpec((8, _HEADS * _H), lambda i: (0, 0)),
    ],
    out_specs=[
        pl.BlockSpec((_HEADS, _NB, _H), lambda i: (0, i, 0)),
        pl.BlockSpec((8, _NB), lambda i: (0, i)),
        pl.BlockSpec((8, _NB), lambda i: (0, i)),
        pl.BlockSpec((8, 128), lambda i: (0, 0)),
    ],
    out_shape=[
        jax.ShapeDtypeStruct((_HEADS, _NP, _H), jnp.float32),
        jax.ShapeDtypeStruct((8, _NP), jnp.float32),
        jax.ShapeDtypeStruct((8, _NP), jnp.float32),
        jax.ShapeDtypeStruct((8, 128), jnp.float32),
    ],
)


def _tc2_body(fa_ref, da_ref, fb_ref, db_ref, b1_ref, w2_ref, ast_ref, adt_ref,
              htab_ref, as_ref, ad_ref, sm_ref):
    i = pl.program_id(0)
    xs = []
    for hd in range(_HEADS):
        f_ref, d_ref = (fa_ref, da_ref) if hd < 2 else (fb_ref, db_ref)
        num = f_ref[hd % 2]
        den = d_ref[hd % 2][:, 0:1]
        v = num / (den + 1e-16) + b1_ref[0:1, hd * _H:(hd + 1) * _H]
        xs.append(_elu(v))
    x2 = jnp.concatenate(xs, axis=1)
    h2 = jnp.dot(x2, w2_ref[...], preferred_element_type=jnp.float32)
    asT = lax.dot_general(ast_ref[...], h2, (((1,), (1,)), ((), ())),
                          preferred_element_type=jnp.float32)
    adT = lax.dot_general(adt_ref[...], h2, (((1,), (1,)), ((), ())),
                          preferred_element_type=jnp.float32)
    gidx = i * _NB + lax.broadcasted_iota(jnp.int32, (8, _NB), 1)
    valid = gidx < _N
    asT = jnp.where(valid, asT, _NEG)
    adT = jnp.where(valid, adT, 0.0)
    as_ref[...] = asT
    ad_ref[...] = adT
    htab_ref[...] = h2
    sa = jnp.max(asT, axis=1)
    sd = jnp.max(adT, axis=1)
    cur = jnp.concatenate([jnp.broadcast_to(sa[:, None], (8, 64)),
                           jnp.broadcast_to(sd[:, None], (8, 64))], axis=1)

    @pl.when(i == 0)
    def _():
        sm_ref[...] = cur

    @pl.when(i > 0)
    def _():
        sm_ref[...] = jnp.maximum(sm_ref[...], cur)


_tc2 = pl.pallas_call(
    _tc2_body,
    grid=(_NBN,),
    in_specs=[
        pl.BlockSpec((2, _NB, _H), lambda i: (0, i, 0)),
        pl.BlockSpec((2, _NB, 16), lambda i: (0, i, 0)),
        pl.BlockSpec((2, _NB, _H), lambda i: (0, i, 0)),
        pl.BlockSpec((2, _NB, 16), lambda i: (0, i, 0)),
        pl.BlockSpec((1, _HEADS * _H), lambda i: (0, 0)),
        pl.BlockSpec((_HEADS * _H, _H), lambda i: (0, 0)),
        pl.BlockSpec((8, _H), lambda i: (0, 0)),
        pl.BlockSpec((8, _H), lambda i: (0, 0)),
    ],
    out_specs=[
        pl.BlockSpec((_NB, _H), lambda i: (i, 0)),
        pl.BlockSpec((8, _NB), lambda i: (0, i)),
        pl.BlockSpec((8, _NB), lambda i: (0, i)),
        pl.BlockSpec((8, 128), lambda i: (0, 0)),
    ],
    out_shape=[
        jax.ShapeDtypeStruct((_NP, _H), jnp.float32),
        jax.ShapeDtypeStruct((8, _NP), jnp.float32),
        jax.ShapeDtypeStruct((8, _NP), jnp.float32),
        jax.ShapeDtypeStruct((8, 128), jnp.float32),
    ],
)


def _tc3_body(f_ref, d_ref, b2_ref, batch_ref, po_ref):
    i = pl.program_id(0)
    num = f_ref[0] + f_ref[1]
    den = d_ref[0][:, 0:1] + d_ref[1][:, 0:1]
    o = _elu(num / (den + 1e-16) + b2_ref[0:1, :])
    tail = (lax.broadcasted_iota(jnp.int32, (_NB, 64), 1) == 0)
    oa = jnp.concatenate([o, tail.astype(jnp.float32)], axis=1)
    bt = batch_ref[0, 0, :]
    P = (lax.broadcasted_iota(jnp.int32, (_G, _NB), 0)
         == bt[None, :]).astype(jnp.float32)
    contrib = jnp.dot(P, oa, preferred_element_type=jnp.float32)

    @pl.when(i == 0)
    def _():
        po_ref[...] = contrib

    @pl.when(i > 0)
    def _():
        po_ref[...] = po_ref[...] + contrib

    @pl.when(i == _NBN - 1)
    def _():
        s = po_ref[...]
        cnt = jnp.maximum(s[:, _H:_H + 1], 1.0)
        po_ref[...] = s / cnt


_tc3 = pl.pallas_call(
    _tc3_body,
    grid=(_NBN,),
    in_specs=[
        pl.BlockSpec((2, _NB, _H), lambda i: (0, i, 0)),
        pl.BlockSpec((2, _NB, 16), lambda i: (0, i, 0)),
        pl.BlockSpec((1, _H), lambda i: (0, 0)),
        pl.BlockSpec((1, 1, _NB), lambda i: (i, 0, 0)),
    ],
    out_specs=pl.BlockSpec((_G, 128), lambda i: (0, 0)),
    out_shape=jax.ShapeDtypeStruct((_G, 128), jnp.float32),
)


def _make_sc_layer(head_offset, blocks_per_tile, edge_split):
    rows_per_tile = _NP // _NS
    ngrp = blocks_per_tile // _KP
    mesh = plsc.VectorSubcoreMesh(core_axis_name="c", subcore_axis_name="s",
                                  num_cores=_NC, num_subcores=_NS)

    @functools.partial(
        pl.kernel,
        out_type=[jax.ShapeDtypeStruct((2 * _NP, _H), jnp.float32),
                  jax.ShapeDtypeStruct((2 * _NP, 16), jnp.float32)],
        mesh=mesh,
        compiler_params=pltpu.CompilerParams(needs_layout_passes=False,
                                             use_tc_tiling_on_sc=False),
        scratch_types=[
            pltpu.VMEM((2 * _KP, _EB), jnp.int32),
            pltpu.VMEM((2 * _KP, _EB), jnp.int32),
            pltpu.VMEM((_NP,), jnp.float32),
            pltpu.VMEM((_NP,), jnp.float32),
            pltpu.VMEM((1024,), jnp.float32),
            pltpu.VMEM((_KP * _EB, _H), jnp.bfloat16),
            pltpu.VMEM((2 * _EB, _H), jnp.float32),
            pltpu.VMEM((_KP * _EB, 16), jnp.float32),
            pltpu.VMEM((_KP * _EB,), jnp.float32),
            pltpu.VMEM((_KP, _EB), jnp.int32),
            pltpu.VMEM((_KP, _EB), jnp.int32),
            pltpu.VMEM_SHARED((_NP, _H), jnp.float32),
            pltpu.VMEM_SHARED((_NP, 16), jnp.float32),
            pltpu.SemaphoreType.DMA,
            pltpu.SemaphoreType.DMA,
            pltpu.SemaphoreType.DMA,
        ],
    )
    def sck(src_hbm, dst_hbm, ast_hbm, adt_hbm, sm_hbm, htab_hbm,
            of_hbm, od_hbm,
            src_v, dst_v, as_v, ad_v, sm_v, rows_v, frows_v, den_v, w_v,
            gi_v, si_v, accf_s, accd_s, sem_g, sem_i, sem_s):
        c = lax.axis_index("c")
        s = lax.axis_index("s")
        hg = head_offset if edge_split else head_offset + c
        lanes = lax.iota(jnp.int32, 16)
        zeros16 = jnp.zeros((16,), jnp.float32)
        izeros16 = jnp.zeros((16,), jnp.int32)

        def _zf(b, carry):
            for cc in range(_H // 16):
                frows_v[b, pl.ds(cc * 16, 16)] = zeros16
            return carry
        lax.fori_loop(0, 2 * _EB, _zf, 0)

        def _zd(b, carry):
            den_v[b, pl.ds(0, 16)] = zeros16
            return carry
        lax.fori_loop(0, _KP * _EB, _zd, 0)
        for z in range(rows_per_tile // _EB):
            pltpu.sync_copy(
                frows_v.at[pl.ds(0, _EB)],
                accf_s.at[pl.ds(s * rows_per_tile + z * _EB, _EB)])
        pltpu.sync_copy(den_v.at[pl.ds(0, _KP * _EB)],
                        accd_s.at[pl.ds(s * rows_per_tile, _KP * _EB)])
        pltpu.sync_copy(den_v.at[pl.ds(0, rows_per_tile - _KP * _EB)],
                        accd_s.at[pl.ds(s * rows_per_tile + _KP * _EB,
                                        rows_per_tile - _KP * _EB)])

        pltpu.sync_copy(ast_hbm.at[hg], as_v)
        pltpu.sync_copy(adt_hbm.at[hg], ad_v)
        pltpu.sync_copy(sm_hbm, sm_v)
        plsc.subcore_barrier()

        sa = plsc.load_gather(sm_v, [hg * 128 + lanes])
        sd = plsc.load_gather(sm_v, [hg * 128 + 64 + lanes])
        sv = sa + sd
        sv = jnp.maximum(sv, 0.2 * sv)
        hoff = hg * _NP
        base_blk = s * blocks_per_tile
        if edge_split:
            base_blk = base_blk + c * (_NS * blocks_per_tile)

        pltpu.sync_copy(src_hbm.at[pl.ds(base_blk, _KP)],
                        src_v.at[pl.ds(0, _KP)])
        pltpu.sync_copy(dst_hbm.at[pl.ds(base_blk, _KP)],
                        dst_v.at[pl.ds(0, _KP)])

        def _grp(g, carry):
            po = lax.rem(g, 2) * _KP
            pn = lax.rem(g + 1, 2) * _KP

            @pl.when(g > 0)
            def _():
                pltpu.make_async_copy(
                    src_hbm.at[pl.ds(base_blk, _KP)],
                    src_v.at[pl.ds(po, _KP)], sem_i).wait()
                pltpu.make_async_copy(
                    dst_hbm.at[pl.ds(base_blk, _KP)],
                    dst_v.at[pl.ds(po, _KP)], sem_i).wait()

            @pl.when(g + 1 < ngrp)
            def _():
                nb = base_blk + (g + 1) * _KP
                pltpu.async_copy(src_hbm.at[pl.ds(nb, _KP)],
                                 src_v.at[pl.ds(pn, _KP)], sem_i)
                pltpu.async_copy(dst_hbm.at[pl.ds(nb, _KP)],
                                 dst_v.at[pl.ds(pn, _KP)], sem_i)

            gd = []
            for k in range(_KP):
                for gg in range(_EB // 16):
                    i_s = src_v[po + k, pl.ds(gg * 16, 16)]
                    i_d = dst_v[po + k, pl.ds(gg * 16, 16)]
                    a = plsc.load_gather(as_v, [i_s])
                    b = plsc.load_gather(ad_v, [i_d])
                    z = a + b
                    zl = jnp.maximum(z, 0.2 * z)
                    w = jnp.exp(zl - sv)
                    w_v[pl.ds(k * _EB + gg * 16, 16)] = w
                    plsc.store_scatter(
                        den_v, [k * _EB + gg * 16 + lanes, izeros16], w)
                    gi_v[k, pl.ds(gg * 16, 16)] = i_s + hoff
                    si_v[k, pl.ds(gg * 16, 16)] = i_d
                gd.append(pltpu.async_copy(
                    htab_hbm.at[gi_v.at[k]],
                    rows_v.at[pl.ds(k * _EB, _EB)], sem_g))

            sdl = []
            for k in range(_KP):
                gd[k].wait()
                if k >= 2:
                    sdl[2 * (k - 2)].wait()
                fbase = (k % 2) * _EB

                def _scale2(b2, carry3):
                    wv = plsc.load_gather(
                        w_v, [jnp.full((16,), b2, jnp.int32)])
                    fb = fbase + b2 - k * _EB
                    for cc in range(_H // 32):
                        t = rows_v[b2, pl.ds(cc * 32, 32)]
                        ev, od = plsc.unpack(
                            t, format=plsc.PackFormat.INTERLEAVED)
                        frows_v[fb, pl.ds(cc * 32, 16)] = ev * wv
                        frows_v[fb, pl.ds(cc * 32 + 16, 16)] = od * wv
                    return carry3
                lax.fori_loop(k * _EB, (k + 1) * _EB, _scale2, 0)
                sdl.append(pltpu.async_copy(
                    frows_v.at[pl.ds(fbase, _EB)],
                    accf_s.at[si_v.at[k]], sem_s, add=True))
                sdl.append(pltpu.async_copy(
                    den_v.at[pl.ds(k * _EB, _EB)],
                    accd_s.at[si_v.at[k]], sem_s, add=True))
            for d in sdl[2 * (_KP - 2):]:
                d.wait()
            for k in range(_KP - 2):
                sdl[2 * k + 1].wait()
            return carry
        lax.fori_loop(0, ngrp, _grp, 0)

        plsc.subcore_barrier()
        pltpu.sync_copy(
            accf_s.at[pl.ds(s * rows_per_tile, rows_per_tile)],
            of_hbm.at[pl.ds(c * _NP + s * rows_per_tile, rows_per_tile)])
        pltpu.sync_copy(
            accd_s.at[pl.ds(s * rows_per_tile, rows_per_tile)],
            od_hbm.at[pl.ds(c * _NP + s * rows_per_tile, rows_per_tile)])

    return sck


_sc_cache = {}


def _get_sc(key):
    if key not in _sc_cache:
        if key == "l1a":
            _sc_cache[key] = _make_sc_layer(
                head_offset=0, blocks_per_tile=_EPB1 // _NS, edge_split=False)
        elif key == "l1b":
            _sc_cache[key] = _make_sc_layer(
                head_offset=2, blocks_per_tile=_EPB1 // _NS, edge_split=False)
        else:
            _sc_cache[key] = _make_sc_layer(
                head_offset=0, blocks_per_tile=_EPB2 // (_NS * _NC),
                edge_split=True)
    return _sc_cache[key]


def _prep_edges(ei):
    loop = jnp.arange(_N, dtype=jnp.int32)
    padv = jnp.full((_EPAD - _ET,), _N, jnp.int32)
    src = jnp.concatenate([ei[0], loop, padv])
    dst = jnp.concatenate([ei[1], loop, padv])
    src1 = src[:_EPB1 * _EB].reshape(_EPB1, _EB)
    dst1 = dst[:_EPB1 * _EB].reshape(_EPB1, _EB)
    return (src1, dst1, src.reshape(_EPB2, _EB), dst.reshape(_EPB2, _EB))


def _embed(x, ei, batch, W1, AsT1, AdT1, b1r, W2, AsT2, AdT2, b2r):
    xp = jnp.pad(x, ((0, _NP - _N), (0, 0)))
    srcB1, dstB1, srcB2, dstB2 = _prep_edges(ei)
    htab, asT, adT, smax = _tc1(xp, W1, AsT1, AdT1)
    htab_bf = htab[:, :, _PERM].astype(jnp.bfloat16).reshape(_HEADS * _NP, _H)
    sm = smax.reshape(-1)
    f1a, d1a = _get_sc("l1a")(srcB1, dstB1, asT, adT, sm, htab_bf)
    f1b, d1b = _get_sc("l1b")(srcB1, dstB1, asT, adT, sm, htab_bf)
    htab2, asT2, adT2, smax2 = _tc2(f1a.reshape(2, _NP, _H),
                                    d1a.reshape(2, _NP, 16),
                                    f1b.reshape(2, _NP, _H),
                                    d1b.reshape(2, _NP, 16),
                                    b1r, W2, AsT2, AdT2)
    htab2_bf = htab2[:, _PERM].astype(jnp.bfloat16)
    f2, d2 = _get_sc("l2")(srcB2, dstB2, asT2, adT2, smax2.reshape(-1),
                           htab2_bf)
    batch3 = jnp.concatenate(
        [batch, jnp.full((_NP - _N,), _G, jnp.int32)]).reshape(_NBN, 1, _NB)
    po = _tc3(f2.reshape(2, _NP, _H), d2.reshape(2, _NP, 16), b2r, batch3)
    return po[:_G, :_H]


def _blockdiag_t(att, heads):
    out = jnp.zeros((8, heads * _H), jnp.float32)
    for h in range(heads):
        out = out.at[h, h * _H:(h + 1) * _H].set(att[0, h])
    return out


def kernel(x1, edge_index1, batch1, x2, edge_index2, batch2,
           W1, att_src1, att_dst1, b1, W2, att_src2, att_dst2, b2):
    AsT1 = _blockdiag_t(att_src1, _HEADS)
    AdT1 = _blockdiag_t(att_dst1, _HEADS)
    AsT2 = _blockdiag_t(att_src2, 1)
    AdT2 = _blockdiag_t(att_dst2, 1)
    b1r = b1.reshape(1, _HEADS * _H)
    b2r = b2.reshape(1, _H)
    emb1 = _embed(x1, edge_index1, batch1, W1, AsT1, AdT1, b1r,
                  W2, AsT2, AdT2, b2r)
    emb2 = _embed(x2, edge_index2, batch2, W1, AsT1, AdT1, b1r,
                  W2, AsT2, AdT2, b2r)
    return (emb1, emb2)


# speedup vs baseline: 40.1862x; 1.0199x over previous
import functools

import jax
import jax.numpy as jnp
import numpy as np
from jax import lax
from jax.experimental import pallas as pl
from jax.experimental.pallas import tpu as pltpu
from jax.experimental.pallas import tpu_sc as plsc

_N = 10000
_D = 128
_H = 64
_HEADS = 4
_G = 16

_NP = 10240
_NB = 512
_NBN = _NP // _NB
_EB = 128
_E = 320000
_ET = _E + _N
_EPB1 = 2688
_EPB2 = 2816
_EPAD = _EPB2 * _EB
_NEG = -1e30

_NC = 2
_NS = 16
_KP = 4

_PERM = np.zeros(_H, np.int32)
for _m in range(_H):
    _q, _r = divmod(_m, 32)
    _PERM[_m] = 32 * _q + (_r // 2 if _r % 2 == 0 else 16 + _r // 2)


def _elu(v):
    return jnp.where(v > 0, v, jnp.exp(jnp.minimum(v, 0.0)) - 1.0)


def _tc1_body(x_ref, w_ref, ast_ref, adt_ref, htab_ref, as_ref, ad_ref, sm_ref):
    i = pl.program_id(0)
    h = jnp.dot(x_ref[...], w_ref[...], preferred_element_type=jnp.float32)
    asT = lax.dot_general(ast_ref[...], h, (((1,), (1,)), ((), ())),
                          preferred_element_type=jnp.float32)
    adT = lax.dot_general(adt_ref[...], h, (((1,), (1,)), ((), ())),
                          preferred_element_type=jnp.float32)
    gidx = i * _NB + lax.broadcasted_iota(jnp.int32, (8, _NB), 1)
    valid = gidx < _N
    asT = jnp.where(valid, asT, _NEG)
    adT = jnp.where(valid, adT, 0.0)
    as_ref[...] = asT
    ad_ref[...] = adT
    for hd in range(_HEADS):
        htab_ref[hd, :, :] = h[:, hd * _H:(hd + 1) * _H]
    sa = jnp.max(asT, axis=1)
    sd = jnp.max(adT, axis=1)
    cur = jnp.concatenate([jnp.broadcast_to(sa[:, None], (8, 64)),
                           jnp.broadcast_to(sd[:, None], (8, 64))], axis=1)

    @pl.when(i == 0)
    def _():
        sm_ref[...] = cur

    @pl.when(i > 0)
    def _():
        sm_ref[...] = jnp.maximum(sm_ref[...], cur)


_tc1 = pl.pallas_call(
    _tc1_body,
    grid=(_NBN,),
    in_specs=[
        pl.BlockSpec((_NB, _D), lambda i: (i, 0)),
        pl.BlockSpec((_D, _HEADS * _H), lambda i: (0, 0)),
        pl.BlockSpec((8, _HEADS * _H), lambda i: (0, 0)),
        pl.BlockSpec((8, _HEADS * _H), lambda i: (0, 0)),
    ],
    out_specs=[
        pl.BlockSpec((_HEADS, _NB, _H), lambda i: (0, i, 0)),
        pl.BlockSpec((8, _NB), lambda i: (0, i)),
        pl.BlockSpec((8, _NB), lambda i: (0, i)),
        pl.BlockSpec((8, 128), lambda i: (0, 0)),
    ],
    out_shape=[
        jax.ShapeDtypeStruct((_HEADS, _NP, _H), jnp.float32),
        jax.ShapeDtypeStruct((8, _NP), jnp.float32),
        jax.ShapeDtypeStruct((8, _NP), jnp.float32),
        jax.ShapeDtypeStruct((8, 128), jnp.float32),
    ],
)


def _tc2_body(fa_ref, da_ref, fb_ref, db_ref, b1_ref, w2_ref, ast_ref, adt_ref,
              htab_ref, as_ref, ad_ref, sm_ref):
    i = pl.program_id(0)
    xs = []
    for hd in range(_HEADS):
        f_ref, d_ref = (fa_ref, da_ref) if hd < 2 else (fb_ref, db_ref)
        num = f_ref[hd % 2]
        den = d_ref[hd % 2][:, 0:1]
        v = num / (den + 1e-16) + b1_ref[0:1, hd * _H:(hd + 1) * _H]
        xs.append(_elu(v))
    x2 = jnp.concatenate(xs, axis=1)
    h2 = jnp.dot(x2, w2_ref[...], preferred_element_type=jnp.float32)
    asT = lax.dot_general(ast_ref[...], h2, (((1,), (1,)), ((), ())),
                          preferred_element_type=jnp.float32)
    adT = lax.dot_general(adt_ref[...], h2, (((1,), (1,)), ((), ())),
                          preferred_element_type=jnp.float32)
    gidx = i * _NB + lax.broadcasted_iota(jnp.int32, (8, _NB), 1)
    valid = gidx < _N
    asT = jnp.where(valid, asT, _NEG)
    adT = jnp.where(valid, adT, 0.0)
    as_ref[...] = asT
    ad_ref[...] = adT
    htab_ref[...] = h2
    sa = jnp.max(asT, axis=1)
    sd = jnp.max(adT, axis=1)
    cur = jnp.concatenate([jnp.broadcast_to(sa[:, None], (8, 64)),
                           jnp.broadcast_to(sd[:, None], (8, 64))], axis=1)

    @pl.when(i == 0)
    def _():
        sm_ref[...] = cur

    @pl.when(i > 0)
    def _():
        sm_ref[...] = jnp.maximum(sm_ref[...], cur)


_tc2 = pl.pallas_call(
    _tc2_body,
    grid=(_NBN,),
    in_specs=[
        pl.BlockSpec((2, _NB, _H), lambda i: (0, i, 0)),
        pl.BlockSpec((2, _NB, 16), lambda i: (0, i, 0)),
        pl.BlockSpec((2, _NB, _H), lambda i: (0, i, 0)),
        pl.BlockSpec((2, _NB, 16), lambda i: (0, i, 0)),
        pl.BlockSpec((1, _HEADS * _H), lambda i: (0, 0)),
        pl.BlockSpec((_HEADS * _H, _H), lambda i: (0, 0)),
        pl.BlockSpec((8, _H), lambda i: (0, 0)),
        pl.BlockSpec((8, _H), lambda i: (0, 0)),
    ],
    out_specs=[
        pl.BlockSpec((_NB, _H), lambda i: (i, 0)),
        pl.BlockSpec((8, _NB), lambda i: (0, i)),
        pl.BlockSpec((8, _NB), lambda i: (0, i)),
        pl.BlockSpec((8, 128), lambda i: (0, 0)),
    ],
    out_shape=[
        jax.ShapeDtypeStruct((_NP, _H), jnp.float32),
        jax.ShapeDtypeStruct((8, _NP), jnp.float32),
        jax.ShapeDtypeStruct((8, _NP), jnp.float32),
        jax.ShapeDtypeStruct((8, 128), jnp.float32),
    ],
)


def _tc3_body(f_ref, d_ref, b2_ref, batch_ref, po_ref):
    i = pl.program_id(0)
    num = f_ref[0]
    den = d_ref[0][:, 0:1]
    o = _elu(num / (den + 1e-16) + b2_ref[0:1, :])
    tail = (lax.broadcasted_iota(jnp.int32, (_NB, 64), 1) == 0)
    oa = jnp.concatenate([o, tail.astype(jnp.float32)], axis=1)
    bt = batch_ref[0, 0, :]
    P = (lax.broadcasted_iota(jnp.int32, (_G, _NB), 0)
         == bt[None, :]).astype(jnp.float32)
    contrib = jnp.dot(P, oa, preferred_element_type=jnp.float32)

    @pl.when(i == 0)
    def _():
        po_ref[...] = contrib

    @pl.when(i > 0)
    def _():
        po_ref[...] = po_ref[...] + contrib

    @pl.when(i == _NBN - 1)
    def _():
        s = po_ref[...]
        cnt = jnp.maximum(s[:, _H:_H + 1], 1.0)
        po_ref[...] = s / cnt


_tc3 = pl.pallas_call(
    _tc3_body,
    grid=(_NBN,),
    in_specs=[
        pl.BlockSpec((1, _NB, _H), lambda i: (0, i, 0)),
        pl.BlockSpec((1, _NB, 16), lambda i: (0, i, 0)),
        pl.BlockSpec((1, _H), lambda i: (0, 0)),
        pl.BlockSpec((1, 1, _NB), lambda i: (i, 0, 0)),
    ],
    out_specs=pl.BlockSpec((_G, 128), lambda i: (0, 0)),
    out_shape=jax.ShapeDtypeStruct((_G, 128), jnp.float32),
)


def _make_sc_layer(head_offset, blocks_per_tile, edge_split,
                   core_head=None):
    if core_head is None:
        core_head = not edge_split
    rows_per_tile = _NP // _NS
    ngrp = blocks_per_tile // _KP
    mesh = plsc.VectorSubcoreMesh(core_axis_name="c", subcore_axis_name="s",
                                  num_cores=_NC, num_subcores=_NS)

    @functools.partial(
        pl.kernel,
        out_type=[jax.ShapeDtypeStruct((2 * _NP, _H), jnp.float32),
                  jax.ShapeDtypeStruct((2 * _NP, 16), jnp.float32)],
        mesh=mesh,
        compiler_params=pltpu.CompilerParams(needs_layout_passes=False,
                                             use_tc_tiling_on_sc=False),
        scratch_types=[
            pltpu.VMEM((2 * _KP, _EB), jnp.int32),
            pltpu.VMEM((2 * _KP, _EB), jnp.int32),
            pltpu.VMEM((_NP,), jnp.float32),
            pltpu.VMEM((_NP,), jnp.float32),
            pltpu.VMEM((1024,), jnp.float32),
            pltpu.VMEM((_KP * _EB, _H), jnp.bfloat16),
            pltpu.VMEM((2 * _EB, _H), jnp.float32),
            pltpu.VMEM((_KP * _EB, 16), jnp.float32),
            pltpu.VMEM((_KP * _EB,), jnp.float32),
            pltpu.VMEM((_KP, _EB), jnp.int32),
            pltpu.VMEM((_KP, _EB), jnp.int32),
            pltpu.VMEM_SHARED((_NP, _H), jnp.float32),
            pltpu.VMEM_SHARED((_NP, 16), jnp.float32),
            pltpu.SemaphoreType.DMA,
            pltpu.SemaphoreType.DMA,
            pltpu.SemaphoreType.DMA,
        ],
    )
    def sck(src_hbm, dst_hbm, ast_hbm, adt_hbm, sm_hbm, htab_hbm,
            of_hbm, od_hbm,
            src_v, dst_v, as_v, ad_v, sm_v, rows_v, frows_v, den_v, w_v,
            gi_v, si_v, accf_s, accd_s, sem_g, sem_i, sem_s):
        c = lax.axis_index("c")
        s = lax.axis_index("s")
        hg = head_offset + c if core_head else head_offset
        lanes = lax.iota(jnp.int32, 16)
        zeros16 = jnp.zeros((16,), jnp.float32)
        izeros16 = jnp.zeros((16,), jnp.int32)

        def _zf(b, carry):
            for cc in range(_H // 16):
                frows_v[b, pl.ds(cc * 16, 16)] = zeros16
            return carry
        lax.fori_loop(0, 2 * _EB, _zf, 0)

        def _zd(b, carry):
            den_v[b, pl.ds(0, 16)] = zeros16
            return carry
        lax.fori_loop(0, _KP * _EB, _zd, 0)
        for z in range(rows_per_tile // _EB):
            pltpu.sync_copy(
                frows_v.at[pl.ds(0, _EB)],
                accf_s.at[pl.ds(s * rows_per_tile + z * _EB, _EB)])
        pltpu.sync_copy(den_v.at[pl.ds(0, _KP * _EB)],
                        accd_s.at[pl.ds(s * rows_per_tile, _KP * _EB)])
        pltpu.sync_copy(den_v.at[pl.ds(0, rows_per_tile - _KP * _EB)],
                        accd_s.at[pl.ds(s * rows_per_tile + _KP * _EB,
                                        rows_per_tile - _KP * _EB)])

        pltpu.sync_copy(ast_hbm.at[hg], as_v)
        pltpu.sync_copy(adt_hbm.at[hg], ad_v)
        pltpu.sync_copy(sm_hbm, sm_v)
        plsc.subcore_barrier()

        sa = plsc.load_gather(sm_v, [hg * 128 + lanes])
        sd = plsc.load_gather(sm_v, [hg * 128 + 64 + lanes])
        sv = sa + sd
        sv = jnp.maximum(sv, 0.2 * sv)
        hoff = hg * _NP
        base_blk = s * blocks_per_tile
        if edge_split:
            base_blk = base_blk + c * (_NS * blocks_per_tile)

        pltpu.sync_copy(src_hbm.at[pl.ds(base_blk, _KP)],
                        src_v.at[pl.ds(0, _KP)])
        pltpu.sync_copy(dst_hbm.at[pl.ds(base_blk, _KP)],
                        dst_v.at[pl.ds(0, _KP)])

        def _grp(g, carry):
            po = lax.rem(g, 2) * _KP
            pn = lax.rem(g + 1, 2) * _KP

            @pl.when(g > 0)
            def _():
                pltpu.make_async_copy(
                    src_hbm.at[pl.ds(base_blk, _KP)],
                    src_v.at[pl.ds(po, _KP)], sem_i).wait()
                pltpu.make_async_copy(
                    dst_hbm.at[pl.ds(base_blk, _KP)],
                    dst_v.at[pl.ds(po, _KP)], sem_i).wait()

            @pl.when(g + 1 < ngrp)
            def _():
                nb = base_blk + (g + 1) * _KP
                pltpu.async_copy(src_hbm.at[pl.ds(nb, _KP)],
                                 src_v.at[pl.ds(pn, _KP)], sem_i)
                pltpu.async_copy(dst_hbm.at[pl.ds(nb, _KP)],
                                 dst_v.at[pl.ds(pn, _KP)], sem_i)

            gd = []
            for k in range(_KP):
                for gg in range(_EB // 16):
                    i_s = src_v[po + k, pl.ds(gg * 16, 16)]
                    i_d = dst_v[po + k, pl.ds(gg * 16, 16)]
                    a = plsc.load_gather(as_v, [i_s])
                    b = plsc.load_gather(ad_v, [i_d])
                    z = a + b
                    zl = jnp.maximum(z, 0.2 * z)
                    w = jnp.exp(zl - sv)
                    w_v[pl.ds(k * _EB + gg * 16, 16)] = w
                    plsc.store_scatter(
                        den_v, [k * _EB + gg * 16 + lanes, izeros16], w)
                    gi_v[k, pl.ds(gg * 16, 16)] = i_s + hoff
                    si_v[k, pl.ds(gg * 16, 16)] = i_d
                gd.append(pltpu.async_copy(
                    htab_hbm.at[gi_v.at[k]],
                    rows_v.at[pl.ds(k * _EB, _EB)], sem_g))

            sdl = []
            for k in range(_KP):
                gd[k].wait()
                if k >= 2:
                    sdl[2 * (k - 2)].wait()
                fbase = (k % 2) * _EB

                def _scale2(b2, carry3):
                    wv = plsc.load_gather(
                        w_v, [jnp.full((16,), b2, jnp.int32)])
                    fb = fbase + b2 - k * _EB
                    for cc in range(_H // 32):
                        t = rows_v[b2, pl.ds(cc * 32, 32)]
                        ev, od = plsc.unpack(
                            t, format=plsc.PackFormat.INTERLEAVED)
                        frows_v[fb, pl.ds(cc * 32, 16)] = ev * wv
                        frows_v[fb, pl.ds(cc * 32 + 16, 16)] = od * wv
                    return carry3
                lax.fori_loop(k * _EB, (k + 1) * _EB, _scale2, 0)
                sdl.append(pltpu.async_copy(
                    frows_v.at[pl.ds(fbase, _EB)],
                    accf_s.at[si_v.at[k]], sem_s, add=True))
                sdl.append(pltpu.async_copy(
                    den_v.at[pl.ds(k * _EB, _EB)],
                    accd_s.at[si_v.at[k]], sem_s, add=True))
            for d in sdl[2 * (_KP - 2):]:
                d.wait()
            for k in range(_KP - 2):
                sdl[2 * k + 1].wait()
            return carry
        lax.fori_loop(0, ngrp, _grp, 0)

        plsc.subcore_barrier()
        pltpu.sync_copy(
            accf_s.at[pl.ds(s * rows_per_tile, rows_per_tile)],
            of_hbm.at[pl.ds(c * _NP + s * rows_per_tile, rows_per_tile)])
        pltpu.sync_copy(
            accd_s.at[pl.ds(s * rows_per_tile, rows_per_tile)],
            od_hbm.at[pl.ds(c * _NP + s * rows_per_tile, rows_per_tile)])

    return sck


_sc_cache = {}


def _get_sc(key):
    if key not in _sc_cache:
        if key == "l1a":
            _sc_cache[key] = _make_sc_layer(
                head_offset=0, blocks_per_tile=_EPB1 // _NS, edge_split=False)
        elif key == "l1b":
            _sc_cache[key] = _make_sc_layer(
                head_offset=2, blocks_per_tile=_EPB1 // _NS, edge_split=False)
        elif key == "l2m":
            _sc_cache[key] = _make_sc_layer(
                head_offset=0, blocks_per_tile=_EPB2 // _NS,
                edge_split=True, core_head=True)
        else:
            _sc_cache[key] = _make_sc_layer(
                head_offset=0, blocks_per_tile=_EPB2 // (_NS * _NC),
                edge_split=True)
    return _sc_cache[key]


def _prep_edges(ei):
    loop = jnp.arange(_N, dtype=jnp.int32)
    padv = jnp.full((_EPAD - _ET,), _N, jnp.int32)
    src = jnp.concatenate([ei[0], loop, padv])
    dst = jnp.concatenate([ei[1], loop, padv])
    src1 = src[:_EPB1 * _EB].reshape(_EPB1, _EB)
    dst1 = dst[:_EPB1 * _EB].reshape(_EPB1, _EB)
    return (src1, dst1, src.reshape(_EPB2, _EB), dst.reshape(_EPB2, _EB))


def _embed_front(x, ei, W1, AsT1, AdT1, b1r, W2, AsT2, AdT2):
    xp = jnp.pad(x, ((0, _NP - _N), (0, 0)))
    srcB1, dstB1, srcB2, dstB2 = _prep_edges(ei)
    htab, asT, adT, smax = _tc1(xp, W1, AsT1, AdT1)
    htab_bf = htab[:, :, _PERM].astype(jnp.bfloat16).reshape(_HEADS * _NP, _H)
    sm = smax.reshape(-1)
    f1a, d1a = _get_sc("l1a")(srcB1, dstB1, asT, adT, sm, htab_bf)
    f1b, d1b = _get_sc("l1b")(srcB1, dstB1, asT, adT, sm, htab_bf)
    htab2, asT2, adT2, smax2 = _tc2(f1a.reshape(2, _NP, _H),
                                    d1a.reshape(2, _NP, 16),
                                    f1b.reshape(2, _NP, _H),
                                    d1b.reshape(2, _NP, 16),
                                    b1r, W2, AsT2, AdT2)
    htab2_bf = htab2[:, _PERM].astype(jnp.bfloat16)
    return srcB2, dstB2, htab2_bf, asT2, adT2, smax2


def _batch3(batch):
    return jnp.concatenate(
        [batch, jnp.full((_NP - _N,), _G, jnp.int32)]).reshape(_NBN, 1, _NB)


def _blockdiag_t(att, heads):
    out = jnp.zeros((8, heads * _H), jnp.float32)
    for h in range(heads):
        out = out.at[h, h * _H:(h + 1) * _H].set(att[0, h])
    return out


def kernel(x1, edge_index1, batch1, x2, edge_index2, batch2,
           W1, att_src1, att_dst1, b1, W2, att_src2, att_dst2, b2):
    AsT1 = _blockdiag_t(att_src1, _HEADS)
    AdT1 = _blockdiag_t(att_dst1, _HEADS)
    AsT2 = _blockdiag_t(att_src2, 1)
    AdT2 = _blockdiag_t(att_dst2, 1)
    b1r = b1.reshape(1, _HEADS * _H)
    b2r = b2.reshape(1, _H)
    s1, t1, h1, as1, ad1, sm1 = _embed_front(
        x1, edge_index1, W1, AsT1, AdT1, b1r, W2, AsT2, AdT2)
    s2, t2, h2, as2, ad2, sm2 = _embed_front(
        x2, edge_index2, W1, AsT1, AdT1, b1r, W2, AsT2, AdT2)
    src_m = jnp.concatenate([s1, s2], axis=0)
    dst_m = jnp.concatenate([t1, t2], axis=0)
    ast_m = jnp.concatenate([as1[0:1], as2[0:1]], axis=0)
    adt_m = jnp.concatenate([ad1[0:1], ad2[0:1]], axis=0)
    sm_m = jnp.concatenate(
        [sm1[0], sm2[0], jnp.zeros((768,), jnp.float32)])
    htab_m = jnp.concatenate([h1, h2], axis=0)
    f2, d2 = _get_sc("l2m")(src_m, dst_m, ast_m, adt_m, sm_m, htab_m)
    f2 = f2.reshape(2, _NP, _H)
    d2 = d2.reshape(2, _NP, 16)
    emb1 = _tc3(f2[0:1], d2[0:1], b2r, _batch3(batch1))[:_G, :_H]
    emb2 = _tc3(f2[1:2], d2[1:2], b2r, _batch3(batch2))[:_G, :_H]
    return (emb1, emb2)


# speedup vs baseline: 40.2112x; 1.0006x over previous
import functools

import jax
import jax.numpy as jnp
import numpy as np
from jax import lax
from jax.experimental import pallas as pl
from jax.experimental.pallas import tpu as pltpu
from jax.experimental.pallas import tpu_sc as plsc

_N = 10000
_D = 128
_H = 64
_HEADS = 4
_G = 16

_NP = 10240
_NB = 512
_NBN = _NP // _NB
_EB = 128
_E = 320000
_ET = _E + _N
_EPB1 = 2688
_EPB2 = 2816
_EPAD = _EPB2 * _EB
_NEG = -1e30

_NC = 2
_NS = 16
_KP = 4

_PERM = np.zeros(_H, np.int32)
for _m in range(_H):
    _q, _r = divmod(_m, 32)
    _PERM[_m] = 32 * _q + (_r // 2 if _r % 2 == 0 else 16 + _r // 2)


def _elu(v):
    return jnp.where(v > 0, v, jnp.exp(jnp.minimum(v, 0.0)) - 1.0)


def _tc1_body(x_ref, w_ref, ast_ref, adt_ref, htab_ref, as_ref, ad_ref, sm_ref):
    i = pl.program_id(0)
    h = jnp.dot(x_ref[...], w_ref[...], preferred_element_type=jnp.float32)
    asT = lax.dot_general(ast_ref[...], h, (((1,), (1,)), ((), ())),
                          preferred_element_type=jnp.float32)
    adT = lax.dot_general(adt_ref[...], h, (((1,), (1,)), ((), ())),
                          preferred_element_type=jnp.float32)
    gidx = i * _NB + lax.broadcasted_iota(jnp.int32, (8, _NB), 1)
    valid = gidx < _N
    asT = jnp.where(valid, asT, _NEG)
    adT = jnp.where(valid, adT, 0.0)
    as_ref[...] = asT
    ad_ref[...] = adT
    for hd in range(_HEADS):
        htab_ref[hd, :, :] = h[:, hd * _H:(hd + 1) * _H]
    sa = jnp.max(asT, axis=1)
    sd = jnp.max(adT, axis=1)
    cur = jnp.concatenate([jnp.broadcast_to(sa[:, None], (8, 64)),
                           jnp.broadcast_to(sd[:, None], (8, 64))], axis=1)

    @pl.when(i == 0)
    def _():
        sm_ref[...] = cur

    @pl.when(i > 0)
    def _():
        sm_ref[...] = jnp.maximum(sm_ref[...], cur)


_tc1 = pl.pallas_call(
    _tc1_body,
    grid=(_NBN,),
    in_specs=[
        pl.BlockSpec((_NB, _D), lambda i: (i, 0)),
        pl.BlockSpec((_D, _HEADS * _H), lambda i: (0, 0)),
        pl.BlockSpec((8, _HEADS * _H), lambda i: (0, 0)),
        pl.BlockSpec((8, _HEADS * _H), lambda i: (0, 0)),
    ],
    out_specs=[
        pl.BlockSpec((_HEADS, _NB, _H), lambda i: (0, i, 0)),
        pl.BlockSpec((8, _NB), lambda i: (0, i)),
        pl.BlockSpec((8, _NB), lambda i: (0, i)),
        pl.BlockSpec((8, 128), lambda i: (0, 0)),
    ],
    out_shape=[
        jax.ShapeDtypeStruct((_HEADS, _NP, _H), jnp.float32),
        jax.ShapeDtypeStruct((8, _NP), jnp.float32),
        jax.ShapeDtypeStruct((8, _NP), jnp.float32),
        jax.ShapeDtypeStruct((8, 128), jnp.float32),
    ],
)


def _tc2_body(fa_ref, da_ref, fb_ref, db_ref, b1_ref, w2_ref, ast_ref, adt_ref,
              htab_ref, as_ref, ad_ref, sm_ref):
    i = pl.program_id(0)
    xs = []
    for hd in range(_HEADS):
        f_ref, d_ref = (fa_ref, da_ref) if hd < 2 else (fb_ref, db_ref)
        num = f_ref[hd % 2]
        den = d_ref[hd % 2][:, 0:1]
        v = num / (den + 1e-16) + b1_ref[0:1, hd * _H:(hd + 1) * _H]
        xs.append(_elu(v))
    x2 = jnp.concatenate(xs, axis=1)
    h2 = jnp.dot(x2, w2_ref[...], preferred_element_type=jnp.float32)
    asT = lax.dot_general(ast_ref[...], h2, (((1,), (1,)), ((), ())),
                          preferred_element_type=jnp.float32)
    adT = lax.dot_general(adt_ref[...], h2, (((1,), (1,)), ((), ())),
                          preferred_element_type=jnp.float32)
    gidx = i * _NB + lax.broadcasted_iota(jnp.int32, (8, _NB), 1)
    valid = gidx < _N
    asT = jnp.where(valid, asT, _NEG)
    adT = jnp.where(valid, adT, 0.0)
    as_ref[...] = asT
    ad_ref[...] = adT
    htab_ref[...] = h2
    sa = jnp.max(asT, axis=1)
    sd = jnp.max(adT, axis=1)
    cur = jnp.concatenate([jnp.broadcast_to(sa[:, None], (8, 64)),
                           jnp.broadcast_to(sd[:, None], (8, 64))], axis=1)

    @pl.when(i == 0)
    def _():
        sm_ref[...] = cur

    @pl.when(i > 0)
    def _():
        sm_ref[...] = jnp.maximum(sm_ref[...], cur)


_tc2 = pl.pallas_call(
    _tc2_body,
    grid=(_NBN,),
    in_specs=[
        pl.BlockSpec((2, _NB, _H), lambda i: (0, i, 0)),
        pl.BlockSpec((2, _NB, 16), lambda i: (0, i, 0)),
        pl.BlockSpec((2, _NB, _H), lambda i: (0, i, 0)),
        pl.BlockSpec((2, _NB, 16), lambda i: (0, i, 0)),
        pl.BlockSpec((1, _HEADS * _H), lambda i: (0, 0)),
        pl.BlockSpec((_HEADS * _H, _H), lambda i: (0, 0)),
        pl.BlockSpec((8, _H), lambda i: (0, 0)),
        pl.BlockSpec((8, _H), lambda i: (0, 0)),
    ],
    out_specs=[
        pl.BlockSpec((_NB, _H), lambda i: (i, 0)),
        pl.BlockSpec((8, _NB), lambda i: (0, i)),
        pl.BlockSpec((8, _NB), lambda i: (0, i)),
        pl.BlockSpec((8, 128), lambda i: (0, 0)),
    ],
    out_shape=[
        jax.ShapeDtypeStruct((_NP, _H), jnp.float32),
        jax.ShapeDtypeStruct((8, _NP), jnp.float32),
        jax.ShapeDtypeStruct((8, _NP), jnp.float32),
        jax.ShapeDtypeStruct((8, 128), jnp.float32),
    ],
)


def _tc3_body(f_ref, d_ref, b2_ref, batch_ref, po_ref):
    i = pl.program_id(0)
    num = f_ref[0]
    den = d_ref[0][:, 0:1]
    o = _elu(num / (den + 1e-16) + b2_ref[0:1, :])
    tail = (lax.broadcasted_iota(jnp.int32, (_NB, 64), 1) == 0)
    oa = jnp.concatenate([o, tail.astype(jnp.float32)], axis=1)
    bt = batch_ref[0, 0, :]
    P = (lax.broadcasted_iota(jnp.int32, (_G, _NB), 0)
         == bt[None, :]).astype(jnp.float32)
    contrib = jnp.dot(P, oa, preferred_element_type=jnp.float32)

    @pl.when(i == 0)
    def _():
        po_ref[...] = contrib

    @pl.when(i > 0)
    def _():
        po_ref[...] = po_ref[...] + contrib

    @pl.when(i == _NBN - 1)
    def _():
        s = po_ref[...]
        cnt = jnp.maximum(s[:, _H:_H + 1], 1.0)
        po_ref[...] = s / cnt


_tc3 = pl.pallas_call(
    _tc3_body,
    grid=(_NBN,),
    in_specs=[
        pl.BlockSpec((1, _NB, _H), lambda i: (0, i, 0)),
        pl.BlockSpec((1, _NB, 16), lambda i: (0, i, 0)),
        pl.BlockSpec((1, _H), lambda i: (0, 0)),
        pl.BlockSpec((1, 1, _NB), lambda i: (i, 0, 0)),
    ],
    out_specs=pl.BlockSpec((_G, 128), lambda i: (0, 0)),
    out_shape=jax.ShapeDtypeStruct((_G, 128), jnp.float32),
)


def _make_sc_layer(head_offset, blocks_per_tile, edge_split,
                   core_head=None):
    if core_head is None:
        core_head = not edge_split
    rows_per_tile = _NP // _NS
    ngrp = blocks_per_tile // _KP
    mesh = plsc.VectorSubcoreMesh(core_axis_name="c", subcore_axis_name="s",
                                  num_cores=_NC, num_subcores=_NS)

    @functools.partial(
        pl.kernel,
        out_type=[jax.ShapeDtypeStruct((2 * _NP, _H), jnp.float32),
                  jax.ShapeDtypeStruct((2 * _NP, 16), jnp.float32)],
        mesh=mesh,
        compiler_params=pltpu.CompilerParams(needs_layout_passes=False,
                                             use_tc_tiling_on_sc=False),
        scratch_types=[
            pltpu.VMEM((2 * _KP, _EB), jnp.int32),
            pltpu.VMEM((2 * _KP, _EB), jnp.int32),
            pltpu.VMEM((_NP,), jnp.float32),
            pltpu.VMEM((_NP,), jnp.float32),
            pltpu.VMEM((1024,), jnp.float32),
            pltpu.VMEM((_KP * _EB, _H), jnp.bfloat16),
            pltpu.VMEM((2 * _EB, _H), jnp.float32),
            pltpu.VMEM((_KP * _EB, 16), jnp.float32),
            pltpu.VMEM((_KP * _EB,), jnp.float32),
            pltpu.VMEM((_KP, _EB), jnp.int32),
            pltpu.VMEM((_KP, _EB), jnp.int32),
            pltpu.VMEM_SHARED((_NP, _H), jnp.float32),
            pltpu.VMEM_SHARED((_NP, 16), jnp.float32),
            pltpu.SemaphoreType.DMA,
            pltpu.SemaphoreType.DMA,
            pltpu.SemaphoreType.DMA,
        ],
    )
    def sck(src_hbm, dst_hbm, ast_hbm, adt_hbm, sm_hbm, htab_hbm,
            of_hbm, od_hbm,
            src_v, dst_v, as_v, ad_v, sm_v, rows_v, frows_v, den_v, w_v,
            gi_v, si_v, accf_s, accd_s, sem_g, sem_i, sem_s):
        c = lax.axis_index("c")
        s = lax.axis_index("s")
        hg = head_offset + c if core_head else head_offset
        lanes = lax.iota(jnp.int32, 16)
        zeros16 = jnp.zeros((16,), jnp.float32)
        izeros16 = jnp.zeros((16,), jnp.int32)

        def _zf(b, carry):
            for cc in range(_H // 16):
                frows_v[b, pl.ds(cc * 16, 16)] = zeros16
            return carry
        lax.fori_loop(0, 2 * _EB, _zf, 0)

        def _zd(b, carry):
            den_v[b, pl.ds(0, 16)] = zeros16
            return carry
        lax.fori_loop(0, _KP * _EB, _zd, 0)
        for z in range(rows_per_tile // _EB):
            pltpu.sync_copy(
                frows_v.at[pl.ds(0, _EB)],
                accf_s.at[pl.ds(s * rows_per_tile + z * _EB, _EB)])
        pltpu.sync_copy(den_v.at[pl.ds(0, _KP * _EB)],
                        accd_s.at[pl.ds(s * rows_per_tile, _KP * _EB)])
        pltpu.sync_copy(den_v.at[pl.ds(0, rows_per_tile - _KP * _EB)],
                        accd_s.at[pl.ds(s * rows_per_tile + _KP * _EB,
                                        rows_per_tile - _KP * _EB)])

        pltpu.sync_copy(ast_hbm.at[hg], as_v)
        pltpu.sync_copy(adt_hbm.at[hg], ad_v)
        pltpu.sync_copy(sm_hbm, sm_v)
        plsc.subcore_barrier()

        sa = plsc.load_gather(sm_v, [hg * 128 + lanes])
        sd = plsc.load_gather(sm_v, [hg * 128 + 64 + lanes])
        sv = sa + sd
        sv = jnp.maximum(sv, 0.2 * sv)
        hoff = hg * _NP
        base_blk = s * blocks_per_tile
        if edge_split:
            base_blk = base_blk + c * (_NS * blocks_per_tile)

        pltpu.sync_copy(src_hbm.at[pl.ds(base_blk, _KP)],
                        src_v.at[pl.ds(0, _KP)])
        pltpu.sync_copy(dst_hbm.at[pl.ds(base_blk, _KP)],
                        dst_v.at[pl.ds(0, _KP)])

        def _grp(g, carry):
            po = lax.rem(g, 2) * _KP
            pn = lax.rem(g + 1, 2) * _KP

            @pl.when(g > 0)
            def _():
                pltpu.make_async_copy(
                    src_hbm.at[pl.ds(base_blk, _KP)],
                    src_v.at[pl.ds(po, _KP)], sem_i).wait()
                pltpu.make_async_copy(
                    dst_hbm.at[pl.ds(base_blk, _KP)],
                    dst_v.at[pl.ds(po, _KP)], sem_i).wait()

            @pl.when(g + 1 < ngrp)
            def _():
                nb = base_blk + (g + 1) * _KP
                pltpu.async_copy(src_hbm.at[pl.ds(nb, _KP)],
                                 src_v.at[pl.ds(pn, _KP)], sem_i)
                pltpu.async_copy(dst_hbm.at[pl.ds(nb, _KP)],
                                 dst_v.at[pl.ds(pn, _KP)], sem_i)

            gd = []
            for k in range(_KP):
                for gg in range(_EB // 16):
                    i_s = src_v[po + k, pl.ds(gg * 16, 16)]
                    i_d = dst_v[po + k, pl.ds(gg * 16, 16)]
                    a = plsc.load_gather(as_v, [i_s])
                    b = plsc.load_gather(ad_v, [i_d])
                    z = a + b
                    zl = jnp.maximum(z, 0.2 * z)
                    w = jnp.exp(zl - sv)
                    w_v[pl.ds(k * _EB + gg * 16, 16)] = w
                    plsc.store_scatter(
                        den_v, [k * _EB + gg * 16 + lanes, izeros16], w)
                    gi_v[k, pl.ds(gg * 16, 16)] = i_s + hoff
                    si_v[k, pl.ds(gg * 16, 16)] = i_d
                gd.append(pltpu.async_copy(
                    htab_hbm.at[gi_v.at[k]],
                    rows_v.at[pl.ds(k * _EB, _EB)], sem_g))

            sdl = []
            for k in range(_KP):
                gd[k].wait()
                if k >= 2:
                    sdl[2 * (k - 2)].wait()
                fbase = (k % 2) * _EB

                def _scale2(b2, carry3):
                    wv = plsc.load_gather(
                        w_v, [jnp.full((16,), b2, jnp.int32)])
                    fb = fbase + b2 - k * _EB
                    for cc in range(_H // 32):
                        t = rows_v[b2, pl.ds(cc * 32, 32)]
                        ev, od = plsc.unpack(
                            t, format=plsc.PackFormat.INTERLEAVED)
                        frows_v[fb, pl.ds(cc * 32, 16)] = ev * wv
                        frows_v[fb, pl.ds(cc * 32 + 16, 16)] = od * wv
                    return carry3
                lax.fori_loop(k * _EB, (k + 1) * _EB, _scale2, 0)
                sdl.append(pltpu.async_copy(
                    frows_v.at[pl.ds(fbase, _EB)],
                    accf_s.at[si_v.at[k]], sem_s, add=True))
                sdl.append(pltpu.async_copy(
                    den_v.at[pl.ds(k * _EB, _EB)],
                    accd_s.at[si_v.at[k]], sem_s, add=True))
            for d in sdl[2 * (_KP - 2):]:
                d.wait()
            for k in range(_KP - 2):
                sdl[2 * k + 1].wait()
            return carry
        lax.fori_loop(0, ngrp, _grp, 0)

        plsc.subcore_barrier()
        pltpu.sync_copy(
            accf_s.at[pl.ds(s * rows_per_tile, rows_per_tile)],
            of_hbm.at[pl.ds(c * _NP + s * rows_per_tile, rows_per_tile)])
        pltpu.sync_copy(
            accd_s.at[pl.ds(s * rows_per_tile, rows_per_tile)],
            od_hbm.at[pl.ds(c * _NP + s * rows_per_tile, rows_per_tile)])

    return sck


_sc_cache = {}


def _get_sc(key):
    if key not in _sc_cache:
        if key == "l1a":
            _sc_cache[key] = _make_sc_layer(
                head_offset=0, blocks_per_tile=_EPB1 // _NS, edge_split=False)
        elif key == "l1b":
            _sc_cache[key] = _make_sc_layer(
                head_offset=2, blocks_per_tile=_EPB1 // _NS, edge_split=False)
        else:
            _sc_cache[key] = _make_sc_layer(
                head_offset=0, blocks_per_tile=_EPB2 // _NS,
                edge_split=True, core_head=True)
    return _sc_cache[key]


def _prep_edges(ei):
    loop = jnp.arange(_N, dtype=jnp.int32)
    padv = jnp.full((_EPAD - _ET,), _N, jnp.int32)
    src = jnp.concatenate([ei[0], loop, padv])
    dst = jnp.concatenate([ei[1], loop, padv])
    src1 = src[:_EPB1 * _EB].reshape(_EPB1, _EB)
    dst1 = dst[:_EPB1 * _EB].reshape(_EPB1, _EB)
    return (src1, dst1, src.reshape(_EPB2, _EB), dst.reshape(_EPB2, _EB))


def _embed_front(x, ei, W1, AsT1, AdT1, b1r, W2, AsT2, AdT2):
    xp = jnp.pad(x, ((0, _NP - _N), (0, 0)))
    srcB1, dstB1, srcB2, dstB2 = _prep_edges(ei)
    htab, asT, adT, smax = _tc1(xp, W1, AsT1, AdT1)
    htab_bf = htab[:, :, _PERM].astype(jnp.bfloat16).reshape(_HEADS * _NP, _H)
    sm = smax.reshape(-1)
    f1a, d1a = _get_sc("l1a")(srcB1, dstB1, asT, adT, sm, htab_bf)
    f1b, d1b = _get_sc("l1b")(srcB1, dstB1, asT, adT, sm, htab_bf)
    htab2, asT2, adT2, smax2 = _tc2(f1a.reshape(2, _NP, _H),
                                    d1a.reshape(2, _NP, 16),
                                    f1b.reshape(2, _NP, _H),
                                    d1b.reshape(2, _NP, 16),
                                    b1r, W2, AsT2, AdT2)
    htab2_bf = htab2[:, _PERM].astype(jnp.bfloat16)
    return srcB2, dstB2, htab2_bf, asT2, adT2, smax2


def _batch3(batch):
    return jnp.concatenate(
        [batch, jnp.full((_NP - _N,), _G, jnp.int32)]).reshape(_NBN, 1, _NB)


def _blockdiag_t(att, heads):
    out = jnp.zeros((8, heads * _H), jnp.float32)
    for h in range(heads):
        out = out.at[h, h * _H:(h + 1) * _H].set(att[0, h])
    return out


def kernel(x1, edge_index1, batch1, x2, edge_index2, batch2,
           W1, att_src1, att_dst1, b1, W2, att_src2, att_dst2, b2):
    AsT1 = _blockdiag_t(att_src1, _HEADS)
    AdT1 = _blockdiag_t(att_dst1, _HEADS)
    AsT2 = _blockdiag_t(att_src2, 1)
    AdT2 = _blockdiag_t(att_dst2, 1)
    b1r = b1.reshape(1, _HEADS * _H)
    b2r = b2.reshape(1, _H)
    s1, t1, h1, as1, ad1, sm1 = _embed_front(
        x1, edge_index1, W1, AsT1, AdT1, b1r, W2, AsT2, AdT2)
    s2, t2, h2, as2, ad2, sm2 = _embed_front(
        x2, edge_index2, W1, AsT1, AdT1, b1r, W2, AsT2, AdT2)
    src_m = jnp.concatenate([s1, s2], axis=0)
    dst_m = jnp.concatenate([t1, t2], axis=0)
    ast_m = jnp.concatenate([as1[0:1], as2[0:1]], axis=0)
    adt_m = jnp.concatenate([ad1[0:1], ad2[0:1]], axis=0)
    sm_m = jnp.concatenate(
        [sm1[0], sm2[0], jnp.zeros((768,), jnp.float32)])
    htab_m = jnp.concatenate([h1, h2], axis=0)
    f2, d2 = _get_sc("l2m")(src_m, dst_m, ast_m, adt_m, sm_m, htab_m)
    f2 = f2.reshape(2, _NP, _H)
    d2 = d2.reshape(2, _NP, 16)
    emb1 = _tc3(f2[0:1], d2[0:1], b2r, _batch3(batch1))[:_G, :_H]
    emb2 = _tc3(f2[1:2], d2[1:2], b2r, _batch3(batch2))[:_G, :_H]
    return (emb1, emb2)
